# Initial kernel scaffold; baseline (speedup 1.0000x reference)
import jax
import jax.numpy as jnp
from jax.experimental import pallas as pl


def kernel(x, llc_x, edge_index, params):
    raise NotImplementedError("write your pallas kernel here")


# speedup vs baseline: 7.0504x; 7.0504x over previous
import functools

import jax
import jax.numpy as jnp
from jax import lax
from jax.experimental import pallas as pl
from jax.experimental.pallas import tpu as pltpu
from jax.experimental.pallas import tpu_sc as plsc

N = 10000
E = 160000
D_IN = 128
D = 16
H = 16
HD = 256
NC = 2
NS = 16
LPT = N // NS
RB = N // 5

_CA = 64
_EP = 160768
_NCHA = (_EP // NS) // _CA
_CS = 40
_DC = 208
_NCHS = (E // (NC * NS)) // _CS


def _tc_embed(x, Wl, Wr):
    def body(x_ref, wl_ref, wr_ref, y_ref, xr_ref):
        xb = x_ref[...]
        y_ref[...] = jnp.dot(xb, wl_ref[...], preferred_element_type=jnp.float32)
        xr_ref[...] = jnp.dot(xb, wr_ref[...], preferred_element_type=jnp.float32)

    return pl.pallas_call(
        body,
        grid=(5,),
        in_specs=[
            pl.BlockSpec((RB, D_IN), lambda i: (i, 0)),
            pl.BlockSpec((D_IN, D), lambda i: (0, 0)),
            pl.BlockSpec((D_IN, D), lambda i: (0, 0)),
        ],
        out_specs=[
            pl.BlockSpec((RB, D), lambda i: (i, 0)),
            pl.BlockSpec((RB, D), lambda i: (i, 0)),
        ],
        out_shape=[jax.ShapeDtypeStruct((N, D), jnp.float32)] * 2,
    )(x, Wl, Wr)


def _tc_qkv1(aggy, aggc, xr, bl, Wq, bq, Wk, bk, Wv, bv, Ws, bs):
    def body(a0, a1, c0, c1, xr_ref, bl_ref, wq, bq_, wk, bk_, wv, bv_,
             ws, bs_, oq, ok, ov, os_):
        cnt = jnp.maximum(c0[:, 0:1] + c1[:, 0:1], 1.0)
        h = (a0[...] + a1[...]) / cnt + bl_ref[...] + xr_ref[...]
        q = jnp.dot(h, wq[...], preferred_element_type=jnp.float32) + bq_[...]
        k = jnp.dot(h, wk[...], preferred_element_type=jnp.float32) + bk_[...]
        v = jnp.dot(h, wv[...], preferred_element_type=jnp.float32) + bv_[...]
        oq[0] = q[:, :128]
        oq[1] = q[:, 128:]
        ok[0] = k[:, :128]
        ok[1] = k[:, 128:]
        ov[0] = v[:, :128]
        ov[1] = v[:, 128:]
        os_[...] = jnp.dot(h, ws[...], preferred_element_type=jnp.float32) + bs_[...]

    half = pl.BlockSpec((RB, D), lambda i: (i, 0))
    wspec = pl.BlockSpec((D, HD), lambda i: (0, 0))
    bspec = pl.BlockSpec((1, HD), lambda i: (0, 0))
    big = pl.BlockSpec((2, RB, 128), lambda i: (0, i, 0))
    return pl.pallas_call(
        body,
        grid=(5,),
        in_specs=[half, half, half, half, half,
                  pl.BlockSpec((1, D), lambda i: (0, 0)),
                  wspec, bspec, wspec, bspec, wspec, bspec, wspec, bspec],
        out_specs=[big, big, big, pl.BlockSpec((RB, HD), lambda i: (i, 0))],
        out_shape=[jax.ShapeDtypeStruct((2, N, 128), jnp.float32)] * 3
        + [jax.ShapeDtypeStruct((N, HD), jnp.float32)],
    )(aggy[:N], aggy[N:], aggc[:N], aggc[N:], xr, bl,
      Wq, bq, Wk, bk, Wv, bv, Ws, bs)


def _tc_qkv2(t, llc, Wq, bq, Wk, bk, Wv, bv, Ws, bs):
    def body(t_ref, l_ref, wq, bq_, wk, bk_, wv, bv_, ws, bs_,
             oq, ok, ov, os_):
        tb = t_ref[...]
        lb = l_ref[...]
        q = jnp.dot(lb, wq[...], preferred_element_type=jnp.float32) + bq_[...]
        k = jnp.dot(tb, wk[...], preferred_element_type=jnp.float32) + bk_[...]
        v = jnp.dot(tb, wv[...], preferred_element_type=jnp.float32) + bv_[...]
        oq[0] = q[:, :128]
        oq[1] = q[:, 128:]
        ok[0] = k[:, :128]
        ok[1] = k[:, 128:]
        ov[0] = v[:, :128]
        ov[1] = v[:, 128:]
        os_[...] = jnp.dot(lb, ws[...], preferred_element_type=jnp.float32) + bs_[...]

    half = pl.BlockSpec((RB, D), lambda i: (i, 0))
    wspec = pl.BlockSpec((D, HD), lambda i: (0, 0))
    bspec = pl.BlockSpec((1, HD), lambda i: (0, 0))
    big = pl.BlockSpec((2, RB, 128), lambda i: (0, i, 0))
    return pl.pallas_call(
        body,
        grid=(5,),
        in_specs=[half, half, wspec, bspec, wspec, bspec, wspec, bspec,
                  wspec, bspec],
        out_specs=[big, big, big, pl.BlockSpec((RB, HD), lambda i: (i, 0))],
        out_shape=[jax.ShapeDtypeStruct((2, N, 128), jnp.float32)] * 3
        + [jax.ShapeDtypeStruct((N, HD), jnp.float32)],
    )(t, llc, Wq, bq, Wk, bk, Wv, bv, Ws, bs)


def _tc_cat(att, s, W, b):
    def body(a0, a1, s_ref, w_ref, b_ref, o_ref):
        cols = []
        for a in (a0[...], a1[...]):
            for j in range(8):
                den = a[:, 128 + j:129 + j] + 1e-16
                cols.append(a[:, j * 16:(j + 1) * 16] / den)
        tf = jnp.concatenate(cols, axis=1) + s_ref[...]
        o_ref[...] = (
            jnp.dot(tf, w_ref[...], preferred_element_type=jnp.float32)
            + b_ref[...]
        )

    return pl.pallas_call(
        body,
        grid=(5,),
        in_specs=[
            pl.BlockSpec((RB, 144), lambda i: (i, 0)),
            pl.BlockSpec((RB, 144), lambda i: (i + 5, 0)),
            pl.BlockSpec((RB, HD), lambda i: (i, 0)),
            pl.BlockSpec((HD, D), lambda i: (0, 0)),
            pl.BlockSpec((1, D), lambda i: (0, 0)),
        ],
        out_specs=pl.BlockSpec((RB, D), lambda i: (i, 0)),
        out_shape=jax.ShapeDtypeStruct((N, D), jnp.float32),
    )(att, att, s, W, b)


def _tc_ln(t_pre, g, b):
    def body(t_ref, g_ref, b_ref, o_ref):
        t = t_ref[...]
        m = jnp.mean(t)
        v = jnp.mean((t - m) ** 2)
        o_ref[...] = (t - m) / jnp.sqrt(v + 1e-5) * g_ref[...] + b_ref[...]

    return pl.pallas_call(
        body,
        out_shape=jax.ShapeDtypeStruct((N, D), jnp.float32),
    )(t_pre, g, b)


def _tc_lnff(t2_pre, g, b, Wl, Wr, bl):
    def body(t_ref, g_ref, b_ref, wl, wr, bl_, oz, ozr):
        t = t_ref[...]
        m = jnp.mean(t)
        v = jnp.mean((t - m) ** 2)
        t2 = (t - m) / jnp.sqrt(v + 1e-5) * g_ref[...] + b_ref[...]
        oz[...] = jnp.dot(t2, wl[...], preferred_element_type=jnp.float32)
        ozr[...] = (
            jnp.dot(t2, wr[...], preferred_element_type=jnp.float32) + bl_[...]
        )

    return pl.pallas_call(
        body,
        out_shape=[jax.ShapeDtypeStruct((N, D), jnp.float32)] * 2,
    )(t2_pre, g, b, Wl, Wr, bl)


def _tc_final(aggz, aggc, zr, g, b):
    def body(z0, z1, c0, c1, zr_ref, g_ref, b_ref, o_ref):
        cnt = jnp.maximum(c0[:, 0:1] + c1[:, 0:1], 1.0)
        t = (z0[...] + z1[...]) / cnt + zr_ref[...]
        m = jnp.mean(t)
        v = jnp.mean((t - m) ** 2)
        o_ref[...] = (t - m) / jnp.sqrt(v + 1e-5) * g_ref[...] + b_ref[...]

    return pl.pallas_call(
        body,
        out_shape=jax.ShapeDtypeStruct((N, D), jnp.float32),
    )(aggz[:N], aggz[N:], aggc[:N], aggc[N:], zr, g, b)


def _sc_sage(y, src, dst, with_count):
    mesh = plsc.VectorSubcoreMesh(core_axis_name="c", subcore_axis_name="s", num_cores=NC, num_subcores=NS)
    nout = 2 if with_count else 1
    scratch = [
        pltpu.VMEM((_CS,), jnp.int32),
        pltpu.VMEM((_CS,), jnp.int32),
        pltpu.VMEM((_CS, D), jnp.float32),
        pltpu.VMEM((_DC, D), jnp.float32),
        pltpu.VMEM_SHARED((N, D), jnp.float32),
    ]
    if with_count:
        scratch += [
            pltpu.VMEM((_CS, D), jnp.float32),
            pltpu.VMEM_SHARED((N, D), jnp.float32),
        ]

    @functools.partial(
        pl.kernel,
        out_type=[jax.ShapeDtypeStruct((NC * N, D), jnp.float32)] * nout,
        mesh=mesh,
        scratch_types=scratch,
        compiler_params=pltpu.CompilerParams(use_tc_tiling_on_sc=False, needs_layout_passes=False),
    )
    def kfn(y_hbm, src_hbm, dst_hbm, *rest):
        if with_count:
            outy, outc, si, di, yb, db, accy, cb, accc = rest
        else:
            outy, si, di, yb, db, accy = rest
            outc = cb = accc = None
        c = lax.axis_index("c")
        s = lax.axis_index("s")
        iot = lax.iota(jnp.int32, D)
        st = s * 624 + jnp.minimum(s, 2) * 8

        def zrow(i, _):
            db[i, pl.ds(0, D)] = jnp.zeros((D,), jnp.float32)
            return 0

        lax.fori_loop(0, _DC, zrow, 0)

        def zcp(g, _):
            r0 = st + g * _DC
            pltpu.sync_copy(db, accy.at[pl.ds(r0, _DC)])
            if with_count:
                pltpu.sync_copy(db, accc.at[pl.ds(r0, _DC)])
            return 0

        lax.fori_loop(0, 3, zcp, 0)

        @pl.when(s < 2)
        def _():
            r0 = st + 3 * _DC
            pltpu.sync_copy(db.at[pl.ds(0, 8)], accy.at[pl.ds(r0, 8)])
            if with_count:
                pltpu.sync_copy(db.at[pl.ds(0, 8)], accc.at[pl.ds(r0, 8)])

        if with_count:
            onec = jnp.where(iot == 0, 1.0, 0.0).astype(jnp.float32)

            def prep(e, _):
                cb[e, pl.ds(0, D)] = onec
                return 0

            lax.fori_loop(0, _CS, prep, 0)
        plsc.subcore_barrier()

        wid = c * NS + s

        def chunk(i, _):
            base = wid * (E // (NC * NS)) + i * _CS
            pltpu.sync_copy(src_hbm.at[pl.ds(base, _CS)], si)
            pltpu.sync_copy(dst_hbm.at[pl.ds(base, _CS)], di)
            pltpu.sync_copy(y_hbm.at[si], yb)
            pltpu.sync_copy(yb, accy.at[di], add=True)
            if with_count:
                pltpu.sync_copy(cb, accc.at[di], add=True)
            return 0

        lax.fori_loop(0, _NCHS, chunk, 0)
        plsc.subcore_barrier()

        def drain(g, _):
            r0 = st + g * _DC
            pltpu.sync_copy(accy.at[pl.ds(r0, _DC)], outy.at[pl.ds(c * N + r0, _DC)])
            if with_count:
                pltpu.sync_copy(accc.at[pl.ds(r0, _DC)], outc.at[pl.ds(c * N + r0, _DC)])
            return 0

        lax.fori_loop(0, 3, drain, 0)

        @pl.when(s < 2)
        def _():
            r0 = st + 3 * _DC
            pltpu.sync_copy(accy.at[pl.ds(r0, 8)], outy.at[pl.ds(c * N + r0, 8)])
            if with_count:
                pltpu.sync_copy(accc.at[pl.ds(r0, 8)], outc.at[pl.ds(c * N + r0, 8)])

    return kfn(y, src, dst)


def _sc_attn(q, k, v, src, dst):
    mesh = plsc.VectorSubcoreMesh(core_axis_name="c", subcore_axis_name="s", num_cores=NC, num_subcores=NS)

    @functools.partial(
        pl.kernel,
        out_type=jax.ShapeDtypeStruct((NC * N, 144), jnp.float32),
        mesh=mesh,
        scratch_types=[
            pltpu.VMEM((_CA,), jnp.int32),
            pltpu.VMEM((_CA,), jnp.int32),
            pltpu.VMEM((_CA,), jnp.int32),
            pltpu.VMEM((_CA, 128), jnp.float32),
            pltpu.VMEM((_CA, 128), jnp.float32),
            pltpu.VMEM((_CA, 128), jnp.float32),
            pltpu.VMEM((_CA, 144), jnp.float32),
            pltpu.VMEM_SHARED((N + 16, 144), jnp.float32),
        ],
        compiler_params=pltpu.CompilerParams(use_tc_tiling_on_sc=False, needs_layout_passes=False),
    )
    def kfn(q_hbm, k_hbm, v_hbm, src_hbm, dst_hbm, out_hbm,
            si, di, dgi, qb, kb, vb, stg, acc):
        c = lax.axis_index("c")
        s = lax.axis_index("s")
        cN = c * N
        iot = lax.iota(jnp.int32, 16)
        st = s * 624 + jnp.minimum(s, 2) * 8

        def zrow(i, _):
            for j in range(9):
                stg[i, pl.ds(j * 16, 16)] = jnp.zeros((16,), jnp.float32)
            return 0

        lax.fori_loop(0, _CA, zrow, 0)

        def zcp(g, _):
            pltpu.sync_copy(stg.at[pl.ds(0, 48)], acc.at[pl.ds(st + g * 48, 48)])
            return 0

        lax.fori_loop(0, 13, zcp, 0)

        @pl.when(s < 2)
        def _():
            pltpu.sync_copy(stg.at[pl.ds(0, 8)], acc.at[pl.ds(st + 624, 8)])

        @pl.when(s == 0)
        def _():
            pltpu.sync_copy(stg.at[pl.ds(0, 16)], acc.at[pl.ds(N, 16)])

        plsc.subcore_barrier()

        def chunk(i, _):
            base = s * (_EP // NS) + i * _CA
            pltpu.sync_copy(src_hbm.at[pl.ds(base, _CA)], si)
            pltpu.sync_copy(dst_hbm.at[pl.ds(base, _CA)], di)

            def addc(j, _):
                sl = pl.ds(j * 16, 16)
                si[sl] = si[sl] + cN
                dgi[sl] = jnp.minimum(di[sl], N - 1) + cN
                return 0

            lax.fori_loop(0, _CA // 16, addc, 0)
            pltpu.sync_copy(q_hbm.at[dgi], qb)
            pltpu.sync_copy(k_hbm.at[si], kb)
            pltpu.sync_copy(v_hbm.at[si], vb)

            def edge(e, _):
                exv = jnp.zeros((16,), jnp.float32)
                for j in range(8):
                    sl = pl.ds(j * 16, 16)
                    lg = jnp.sum(qb[e, sl] * kb[e, sl]) * 0.25
                    ev = jnp.exp(jnp.broadcast_to(lg, (16,)))
                    stg[e, sl] = vb[e, sl] * ev
                    exv = jnp.where(iot == j, ev, exv)
                stg[e, pl.ds(128, 16)] = exv
                return 0

            lax.fori_loop(0, _CA, edge, 0)
            pltpu.sync_copy(stg, acc.at[di], add=True)
            return 0

        lax.fori_loop(0, _NCHA, chunk, 0)
        plsc.subcore_barrier()

        def drain(g, _):
            r0 = st + g * _DC
            pltpu.sync_copy(acc.at[pl.ds(r0, _DC)], out_hbm.at[pl.ds(cN + r0, _DC)])
            return 0

        lax.fori_loop(0, 3, drain, 0)

        @pl.when(s < 2)
        def _():
            r0 = st + 3 * _DC
            pltpu.sync_copy(acc.at[pl.ds(r0, 8)], out_hbm.at[pl.ds(cN + r0, 8)])

    return kfn(q, k, v, src, dst)


def kernel(x, llc_x, edge_index, params):
    p = params
    src = edge_index[0].astype(jnp.int32)
    dst = edge_index[1].astype(jnp.int32)
    pad = _EP - E
    src_p = jnp.concatenate([src, jnp.zeros((pad,), jnp.int32)])
    dst_p = jnp.concatenate([dst, jnp.full((pad,), N, jnp.int32)])

    def b2(a):
        return a.reshape(1, -1)

    y, xr = _tc_embed(x, p['emb_Wl'], p['emb_Wr'])
    aggy, aggc = _sc_sage(y, src, dst, with_count=True)
    a1 = p['a1']
    q1, k1, v1, s1 = _tc_qkv1(
        aggy, aggc, xr, b2(p['emb_bl']),
        a1['Wq'], b2(a1['bq']), a1['Wk'], b2(a1['bk']),
        a1['Wv'], b2(a1['bv']), a1['Ws'], b2(a1['bs']))
    att1 = _sc_attn(q1.reshape(NC * N, 128), k1.reshape(NC * N, 128),
                    v1.reshape(NC * N, 128), src_p, dst_p)
    t_pre = _tc_cat(att1, s1, p['cat1_W'], b2(p['cat1_b']))
    t = _tc_ln(t_pre, b2(p['ln1_g']), b2(p['ln1_b']))

    a2 = p['a2']
    q2, k2, v2, s2 = _tc_qkv2(
        t, llc_x,
        a2['Wq'], b2(a2['bq']), a2['Wk'], b2(a2['bk']),
        a2['Wv'], b2(a2['bv']), a2['Ws'], b2(a2['bs']))
    att2 = _sc_attn(q2.reshape(NC * N, 128), k2.reshape(NC * N, 128),
                    v2.reshape(NC * N, 128), src_p, dst_p)
    t2_pre = _tc_cat(att2, s2, p['cat2_W'], b2(p['cat2_b']))
    z, zr = _tc_lnff(t2_pre, b2(p['ln2_g']), b2(p['ln2_b']),
                     p['ff_Wl'], p['ff_Wr'], b2(p['ff_bl']))

    (aggz,) = _sc_sage(z, src, dst, with_count=False)
    return _tc_final(aggz, aggc, zr, b2(p['ln3_g']), b2(p['ln3_b']))


# speedup vs baseline: 20.0667x; 2.8462x over previous
import functools

import jax
import jax.numpy as jnp
from jax import lax
from jax.experimental import pallas as pl
from jax.experimental.pallas import tpu as pltpu
from jax.experimental.pallas import tpu_sc as plsc

N = 10000
E = 160000
D_IN = 128
D = 16
H = 16
HD = 256
NC = 2
NS = 16
LPT = N // NS
RB = N // 5

_CA = 64
_EP = 160768
_NCHA = (_EP // NS) // _CA
_CS = 40
_DC = 208
_NCHS = (E // (NC * NS)) // _CS


def _tc_embed(x, Wl, Wr):
    def body(x_ref, wl_ref, wr_ref, y_ref, xr_ref):
        xb = x_ref[...]
        y_ref[...] = jnp.dot(xb, wl_ref[...], preferred_element_type=jnp.float32)
        xr_ref[...] = jnp.dot(xb, wr_ref[...], preferred_element_type=jnp.float32)

    return pl.pallas_call(
        body,
        grid=(5,),
        in_specs=[
            pl.BlockSpec((RB, D_IN), lambda i: (i, 0)),
            pl.BlockSpec((D_IN, D), lambda i: (0, 0)),
            pl.BlockSpec((D_IN, D), lambda i: (0, 0)),
        ],
        out_specs=[
            pl.BlockSpec((RB, D), lambda i: (i, 0)),
            pl.BlockSpec((RB, D), lambda i: (i, 0)),
        ],
        out_shape=[jax.ShapeDtypeStruct((N, D), jnp.float32)] * 2,
    )(x, Wl, Wr)


def _tc_qkv1(aggy, aggc, xr, bl, Wq, bq, Wk, bk, Wv, bv, Ws, bs):
    def body(a0, a1, c0, c1, xr_ref, bl_ref, wq, bq_, wk, bk_, wv, bv_,
             ws, bs_, oq, ok, ov, os_):
        cnt = jnp.maximum(c0[:, 0:1] + c1[:, 0:1], 1.0)
        h = (a0[...] + a1[...]) / cnt + bl_ref[...] + xr_ref[...]
        q = (jnp.dot(h, wq[...], preferred_element_type=jnp.float32)
             + bq_[...]) * 0.25
        k = jnp.dot(h, wk[...], preferred_element_type=jnp.float32) + bk_[...]
        v = jnp.dot(h, wv[...], preferred_element_type=jnp.float32) + bv_[...]
        oq[0] = q[:, :128]
        oq[1] = q[:, 128:]
        ok[0] = k[:, :128]
        ok[1] = k[:, 128:]
        ov[0] = v[:, :128]
        ov[1] = v[:, 128:]
        os_[...] = jnp.dot(h, ws[...], preferred_element_type=jnp.float32) + bs_[...]

    half = pl.BlockSpec((RB, D), lambda i: (i, 0))
    wspec = pl.BlockSpec((D, HD), lambda i: (0, 0))
    bspec = pl.BlockSpec((1, HD), lambda i: (0, 0))
    big = pl.BlockSpec((2, RB, 128), lambda i: (0, i, 0))
    return pl.pallas_call(
        body,
        grid=(5,),
        in_specs=[half, half, half, half, half,
                  pl.BlockSpec((1, D), lambda i: (0, 0)),
                  wspec, bspec, wspec, bspec, wspec, bspec, wspec, bspec],
        out_specs=[big, big, big, pl.BlockSpec((RB, HD), lambda i: (i, 0))],
        out_shape=[jax.ShapeDtypeStruct((2, N, 128), jnp.float32)] * 3
        + [jax.ShapeDtypeStruct((N, HD), jnp.float32)],
    )(aggy[:N], aggy[N:], aggc[:N], aggc[N:], xr, bl,
      Wq, bq, Wk, bk, Wv, bv, Ws, bs)


def _tc_qkv2(t, llc, Wq, bq, Wk, bk, Wv, bv, Ws, bs):
    def body(t_ref, l_ref, wq, bq_, wk, bk_, wv, bv_, ws, bs_,
             oq, ok, ov, os_):
        tb = t_ref[...]
        lb = l_ref[...]
        q = (jnp.dot(lb, wq[...], preferred_element_type=jnp.float32)
             + bq_[...]) * 0.25
        k = jnp.dot(tb, wk[...], preferred_element_type=jnp.float32) + bk_[...]
        v = jnp.dot(tb, wv[...], preferred_element_type=jnp.float32) + bv_[...]
        oq[0] = q[:, :128]
        oq[1] = q[:, 128:]
        ok[0] = k[:, :128]
        ok[1] = k[:, 128:]
        ov[0] = v[:, :128]
        ov[1] = v[:, 128:]
        os_[...] = jnp.dot(lb, ws[...], preferred_element_type=jnp.float32) + bs_[...]

    half = pl.BlockSpec((RB, D), lambda i: (i, 0))
    wspec = pl.BlockSpec((D, HD), lambda i: (0, 0))
    bspec = pl.BlockSpec((1, HD), lambda i: (0, 0))
    big = pl.BlockSpec((2, RB, 128), lambda i: (0, i, 0))
    return pl.pallas_call(
        body,
        grid=(5,),
        in_specs=[half, half, wspec, bspec, wspec, bspec, wspec, bspec,
                  wspec, bspec],
        out_specs=[big, big, big, pl.BlockSpec((RB, HD), lambda i: (i, 0))],
        out_shape=[jax.ShapeDtypeStruct((2, N, 128), jnp.float32)] * 3
        + [jax.ShapeDtypeStruct((N, HD), jnp.float32)],
    )(t, llc, Wq, bq, Wk, bk, Wv, bv, Ws, bs)


def _tc_cat(att, s, W, b):
    def body(a0, a1, s_ref, w_ref, b_ref, o_ref):
        cols = []
        for a in (a0[...], a1[...]):
            for j in range(8):
                den = a[:, 128 + j:129 + j] + 1e-16
                cols.append(a[:, j * 16:(j + 1) * 16] / den)
        tf = jnp.concatenate(cols, axis=1) + s_ref[...]
        o_ref[...] = (
            jnp.dot(tf, w_ref[...], preferred_element_type=jnp.float32)
            + b_ref[...]
        )

    return pl.pallas_call(
        body,
        grid=(5,),
        in_specs=[
            pl.BlockSpec((RB, 144), lambda i: (i, 0)),
            pl.BlockSpec((RB, 144), lambda i: (i + 5, 0)),
            pl.BlockSpec((RB, HD), lambda i: (i, 0)),
            pl.BlockSpec((HD, D), lambda i: (0, 0)),
            pl.BlockSpec((1, D), lambda i: (0, 0)),
        ],
        out_specs=pl.BlockSpec((RB, D), lambda i: (i, 0)),
        out_shape=jax.ShapeDtypeStruct((N, D), jnp.float32),
    )(att, att, s, W, b)


def _tc_ln(t_pre, g, b):
    def body(t_ref, g_ref, b_ref, o_ref):
        t = t_ref[...]
        m = jnp.mean(t)
        v = jnp.mean((t - m) ** 2)
        o_ref[...] = (t - m) / jnp.sqrt(v + 1e-5) * g_ref[...] + b_ref[...]

    return pl.pallas_call(
        body,
        out_shape=jax.ShapeDtypeStruct((N, D), jnp.float32),
    )(t_pre, g, b)


def _tc_lnff(t2_pre, g, b, Wl, Wr, bl):
    def body(t_ref, g_ref, b_ref, wl, wr, bl_, oz, ozr):
        t = t_ref[...]
        m = jnp.mean(t)
        v = jnp.mean((t - m) ** 2)
        t2 = (t - m) / jnp.sqrt(v + 1e-5) * g_ref[...] + b_ref[...]
        oz[...] = jnp.dot(t2, wl[...], preferred_element_type=jnp.float32)
        ozr[...] = (
            jnp.dot(t2, wr[...], preferred_element_type=jnp.float32) + bl_[...]
        )

    return pl.pallas_call(
        body,
        out_shape=[jax.ShapeDtypeStruct((N, D), jnp.float32)] * 2,
    )(t2_pre, g, b, Wl, Wr, bl)


def _tc_final(aggz, aggc, zr, g, b):
    def body(z0, z1, c0, c1, zr_ref, g_ref, b_ref, o_ref):
        cnt = jnp.maximum(c0[:, 0:1] + c1[:, 0:1], 1.0)
        t = (z0[...] + z1[...]) / cnt + zr_ref[...]
        m = jnp.mean(t)
        v = jnp.mean((t - m) ** 2)
        o_ref[...] = (t - m) / jnp.sqrt(v + 1e-5) * g_ref[...] + b_ref[...]

    return pl.pallas_call(
        body,
        out_shape=jax.ShapeDtypeStruct((N, D), jnp.float32),
    )(aggz[:N], aggz[N:], aggc[:N], aggc[N:], zr, g, b)


def _sc_sage(y, src, dst, with_count):
    mesh = plsc.VectorSubcoreMesh(core_axis_name="c", subcore_axis_name="s", num_cores=NC, num_subcores=NS)
    nout = 2 if with_count else 1
    scratch = [
        pltpu.VMEM((_CS,), jnp.int32),
        pltpu.VMEM((_CS,), jnp.int32),
        pltpu.VMEM((_CS, D), jnp.float32),
        pltpu.VMEM((_DC, D), jnp.float32),
        pltpu.VMEM_SHARED((N, D), jnp.float32),
    ]
    if with_count:
        scratch += [
            pltpu.VMEM((_CS, D), jnp.float32),
            pltpu.VMEM_SHARED((N, D), jnp.float32),
        ]

    @functools.partial(
        pl.kernel,
        out_type=[jax.ShapeDtypeStruct((NC * N, D), jnp.float32)] * nout,
        mesh=mesh,
        scratch_types=scratch,
        compiler_params=pltpu.CompilerParams(use_tc_tiling_on_sc=False, needs_layout_passes=False),
    )
    def kfn(y_hbm, src_hbm, dst_hbm, *rest):
        if with_count:
            outy, outc, si, di, yb, db, accy, cb, accc = rest
        else:
            outy, si, di, yb, db, accy = rest
            outc = cb = accc = None
        c = lax.axis_index("c")
        s = lax.axis_index("s")
        iot = lax.iota(jnp.int32, D)
        st = s * 624 + jnp.minimum(s, 2) * 8

        def zrow(i, _):
            db[i, pl.ds(0, D)] = jnp.zeros((D,), jnp.float32)
            return 0

        lax.fori_loop(0, _DC, zrow, 0)

        def zcp(g, _):
            r0 = st + g * _DC
            pltpu.sync_copy(db, accy.at[pl.ds(r0, _DC)])
            if with_count:
                pltpu.sync_copy(db, accc.at[pl.ds(r0, _DC)])
            return 0

        lax.fori_loop(0, 3, zcp, 0)

        @pl.when(s < 2)
        def _():
            r0 = st + 3 * _DC
            pltpu.sync_copy(db.at[pl.ds(0, 8)], accy.at[pl.ds(r0, 8)])
            if with_count:
                pltpu.sync_copy(db.at[pl.ds(0, 8)], accc.at[pl.ds(r0, 8)])

        if with_count:
            onec = jnp.where(iot == 0, 1.0, 0.0).astype(jnp.float32)

            def prep(e, _):
                cb[e, pl.ds(0, D)] = onec
                return 0

            lax.fori_loop(0, _CS, prep, 0)
        plsc.subcore_barrier()

        wid = c * NS + s

        def chunk(i, _):
            base = wid * (E // (NC * NS)) + i * _CS
            pltpu.sync_copy(src_hbm.at[pl.ds(base, _CS)], si)
            pltpu.sync_copy(dst_hbm.at[pl.ds(base, _CS)], di)
            pltpu.sync_copy(y_hbm.at[si], yb)
            pltpu.sync_copy(yb, accy.at[di], add=True)
            if with_count:
                pltpu.sync_copy(cb, accc.at[di], add=True)
            return 0

        lax.fori_loop(0, _NCHS, chunk, 0)
        plsc.subcore_barrier()

        def drain(g, _):
            r0 = st + g * _DC
            pltpu.sync_copy(accy.at[pl.ds(r0, _DC)], outy.at[pl.ds(c * N + r0, _DC)])
            if with_count:
                pltpu.sync_copy(accc.at[pl.ds(r0, _DC)], outc.at[pl.ds(c * N + r0, _DC)])
            return 0

        lax.fori_loop(0, 3, drain, 0)

        @pl.when(s < 2)
        def _():
            r0 = st + 3 * _DC
            pltpu.sync_copy(accy.at[pl.ds(r0, 8)], outy.at[pl.ds(c * N + r0, 8)])
            if with_count:
                pltpu.sync_copy(accc.at[pl.ds(r0, 8)], outc.at[pl.ds(c * N + r0, 8)])

    return kfn(y, src, dst)


def _sc_attn(q, k, v, src, dst):
    mesh = plsc.VectorSubcoreMesh(core_axis_name="c", subcore_axis_name="s", num_cores=NC, num_subcores=NS)

    @functools.partial(
        pl.kernel,
        out_type=jax.ShapeDtypeStruct((NC * N, 144), jnp.float32),
        mesh=mesh,
        scratch_types=[
            pltpu.VMEM((_CA,), jnp.int32),
            pltpu.VMEM((_CA,), jnp.int32),
            pltpu.VMEM((_CA,), jnp.int32),
            pltpu.VMEM((_CA, 128), jnp.float32),
            pltpu.VMEM((_CA, 128), jnp.float32),
            pltpu.VMEM((_CA, 128), jnp.float32),
            pltpu.VMEM((_CA, 144), jnp.float32),
            pltpu.VMEM_SHARED((N + 16, 144), jnp.float32),
        ],
        compiler_params=pltpu.CompilerParams(use_tc_tiling_on_sc=False, needs_layout_passes=False),
    )
    def kfn(q_hbm, k_hbm, v_hbm, src_hbm, dst_hbm, out_hbm,
            si, di, dgi, qb, kb, vb, stg, acc):
        c = lax.axis_index("c")
        s = lax.axis_index("s")
        cN = c * N
        iot = lax.iota(jnp.int32, 16)
        st = s * 624 + jnp.minimum(s, 2) * 8

        def zrow(i, _):
            for j in range(9):
                stg[i, pl.ds(j * 16, 16)] = jnp.zeros((16,), jnp.float32)
            return 0

        lax.fori_loop(0, _CA, zrow, 0)

        def zcp(g, _):
            pltpu.sync_copy(stg.at[pl.ds(0, 48)], acc.at[pl.ds(st + g * 48, 48)])
            return 0

        lax.fori_loop(0, 13, zcp, 0)

        @pl.when(s < 2)
        def _():
            pltpu.sync_copy(stg.at[pl.ds(0, 8)], acc.at[pl.ds(st + 624, 8)])

        @pl.when(s == 0)
        def _():
            pltpu.sync_copy(stg.at[pl.ds(0, 16)], acc.at[pl.ds(N, 16)])

        plsc.subcore_barrier()

        def chunk(i, _):
            base = s * (_EP // NS) + i * _CA
            pltpu.sync_copy(src_hbm.at[pl.ds(base, _CA)], si)
            pltpu.sync_copy(dst_hbm.at[pl.ds(base, _CA)], di)

            def addc(j, _):
                sl = pl.ds(j * 16, 16)
                si[sl] = si[sl] + cN
                dgi[sl] = jnp.minimum(di[sl], N - 1) + cN
                return 0

            lax.fori_loop(0, _CA // 16, addc, 0)
            pltpu.sync_copy(q_hbm.at[dgi], qb)
            pltpu.sync_copy(k_hbm.at[si], kb)
            pltpu.sync_copy(v_hbm.at[si], vb)

            @plsc.parallel_loop(0, _CA, step=1, unroll=4)
            def edge(e):
                exv = jnp.zeros((16,), jnp.float32)
                for j in range(8):
                    sl = pl.ds(j * 16, 16)
                    lg = jnp.sum(qb[e, sl] * kb[e, sl])
                    ev = jnp.exp(jnp.broadcast_to(lg, (16,)))
                    stg[e, sl] = vb[e, sl] * ev
                    exv = jnp.where(iot == j, ev, exv)
                stg[e, pl.ds(128, 16)] = exv
            pltpu.sync_copy(stg, acc.at[di], add=True)
            return 0

        lax.fori_loop(0, _NCHA, chunk, 0)
        plsc.subcore_barrier()

        def drain(g, _):
            r0 = st + g * _DC
            pltpu.sync_copy(acc.at[pl.ds(r0, _DC)], out_hbm.at[pl.ds(cN + r0, _DC)])
            return 0

        lax.fori_loop(0, 3, drain, 0)

        @pl.when(s < 2)
        def _():
            r0 = st + 3 * _DC
            pltpu.sync_copy(acc.at[pl.ds(r0, 8)], out_hbm.at[pl.ds(cN + r0, 8)])

    return kfn(q, k, v, src, dst)


def kernel(x, llc_x, edge_index, params):
    p = params
    src = edge_index[0].astype(jnp.int32)
    dst = edge_index[1].astype(jnp.int32)
    pad = _EP - E
    src_p = jnp.concatenate([src, jnp.zeros((pad,), jnp.int32)])
    dst_p = jnp.concatenate([dst, jnp.full((pad,), N, jnp.int32)])

    def b2(a):
        return a.reshape(1, -1)

    y, xr = _tc_embed(x, p['emb_Wl'], p['emb_Wr'])
    aggy, aggc = _sc_sage(y, src, dst, with_count=True)
    a1 = p['a1']
    q1, k1, v1, s1 = _tc_qkv1(
        aggy, aggc, xr, b2(p['emb_bl']),
        a1['Wq'], b2(a1['bq']), a1['Wk'], b2(a1['bk']),
        a1['Wv'], b2(a1['bv']), a1['Ws'], b2(a1['bs']))
    att1 = _sc_attn(q1.reshape(NC * N, 128), k1.reshape(NC * N, 128),
                    v1.reshape(NC * N, 128), src_p, dst_p)
    t_pre = _tc_cat(att1, s1, p['cat1_W'], b2(p['cat1_b']))
    t = _tc_ln(t_pre, b2(p['ln1_g']), b2(p['ln1_b']))

    a2 = p['a2']
    q2, k2, v2, s2 = _tc_qkv2(
        t, llc_x,
        a2['Wq'], b2(a2['bq']), a2['Wk'], b2(a2['bk']),
        a2['Wv'], b2(a2['bv']), a2['Ws'], b2(a2['bs']))
    att2 = _sc_attn(q2.reshape(NC * N, 128), k2.reshape(NC * N, 128),
                    v2.reshape(NC * N, 128), src_p, dst_p)
    t2_pre = _tc_cat(att2, s2, p['cat2_W'], b2(p['cat2_b']))
    z, zr = _tc_lnff(t2_pre, b2(p['ln2_g']), b2(p['ln2_b']),
                     p['ff_Wl'], p['ff_Wr'], b2(p['ff_bl']))

    (aggz,) = _sc_sage(z, src, dst, with_count=False)
    return _tc_final(aggz, aggc, zr, b2(p['ln3_g']), b2(p['ln3_b']))


# speedup vs baseline: 27.6350x; 1.3772x over previous
import functools

import jax
import jax.numpy as jnp
from jax import lax
from jax.experimental import pallas as pl
from jax.experimental.pallas import tpu as pltpu
from jax.experimental.pallas import tpu_sc as plsc

N = 10000
E = 160000
D_IN = 128
D = 16
H = 16
HD = 256
NC = 2
NS = 16
LPT = N // NS
RB = N // 5

_CA = 32
_EP = 160768
_NCHA = (_EP // NS) // _CA
_CS = 40
_DC = 208
_NCHS = (E // (NC * NS)) // _CS


def _tc_embed(x, Wl, Wr):
    def body(x_ref, wl_ref, wr_ref, y_ref, xr_ref):
        xb = x_ref[...]
        y_ref[...] = jnp.dot(xb, wl_ref[...], preferred_element_type=jnp.float32)
        xr_ref[...] = jnp.dot(xb, wr_ref[...], preferred_element_type=jnp.float32)

    return pl.pallas_call(
        body,
        grid=(5,),
        in_specs=[
            pl.BlockSpec((RB, D_IN), lambda i: (i, 0)),
            pl.BlockSpec((D_IN, D), lambda i: (0, 0)),
            pl.BlockSpec((D_IN, D), lambda i: (0, 0)),
        ],
        out_specs=[
            pl.BlockSpec((RB, D), lambda i: (i, 0)),
            pl.BlockSpec((RB, D), lambda i: (i, 0)),
        ],
        out_shape=[jax.ShapeDtypeStruct((N, D), jnp.float32)] * 2,
    )(x, Wl, Wr)


def _tc_qkv1(aggy, aggc, xr, bl, Wq, bq, Wk, bk, Wv, bv, Ws, bs):
    def body(a0, a1, c0, c1, xr_ref, bl_ref, wq, bq_, wk, bk_, wv, bv_,
             ws, bs_, oq, ok, ov, os_):
        cnt = jnp.maximum(c0[:, 0:1] + c1[:, 0:1], 1.0)
        h = (a0[...] + a1[...]) / cnt + bl_ref[...] + xr_ref[...]
        q = (jnp.dot(h, wq[...], preferred_element_type=jnp.float32)
             + bq_[...]) * 0.25
        k = jnp.dot(h, wk[...], preferred_element_type=jnp.float32) + bk_[...]
        v = jnp.dot(h, wv[...], preferred_element_type=jnp.float32) + bv_[...]
        oq[0] = q[:, :128]
        oq[1] = q[:, 128:]
        ok[0] = k[:, :128]
        ok[1] = k[:, 128:]
        ov[0] = v[:, :128]
        ov[1] = v[:, 128:]
        os_[...] = jnp.dot(h, ws[...], preferred_element_type=jnp.float32) + bs_[...]

    half = pl.BlockSpec((RB, D), lambda i: (i, 0))
    wspec = pl.BlockSpec((D, HD), lambda i: (0, 0))
    bspec = pl.BlockSpec((1, HD), lambda i: (0, 0))
    big = pl.BlockSpec((2, RB, 128), lambda i: (0, i, 0))
    return pl.pallas_call(
        body,
        grid=(5,),
        in_specs=[half, half, half, half, half,
                  pl.BlockSpec((1, D), lambda i: (0, 0)),
                  wspec, bspec, wspec, bspec, wspec, bspec, wspec, bspec],
        out_specs=[big, big, big, pl.BlockSpec((RB, HD), lambda i: (i, 0))],
        out_shape=[jax.ShapeDtypeStruct((2, N, 128), jnp.float32)] * 3
        + [jax.ShapeDtypeStruct((N, HD), jnp.float32)],
    )(aggy[:N], aggy[N:], aggc[:N], aggc[N:], xr, bl,
      Wq, bq, Wk, bk, Wv, bv, Ws, bs)


def _tc_qkv2(t, llc, Wq, bq, Wk, bk, Wv, bv, Ws, bs):
    def body(t_ref, l_ref, wq, bq_, wk, bk_, wv, bv_, ws, bs_,
             oq, ok, ov, os_):
        tb = t_ref[...]
        lb = l_ref[...]
        q = (jnp.dot(lb, wq[...], preferred_element_type=jnp.float32)
             + bq_[...]) * 0.25
        k = jnp.dot(tb, wk[...], preferred_element_type=jnp.float32) + bk_[...]
        v = jnp.dot(tb, wv[...], preferred_element_type=jnp.float32) + bv_[...]
        oq[0] = q[:, :128]
        oq[1] = q[:, 128:]
        ok[0] = k[:, :128]
        ok[1] = k[:, 128:]
        ov[0] = v[:, :128]
        ov[1] = v[:, 128:]
        os_[...] = jnp.dot(lb, ws[...], preferred_element_type=jnp.float32) + bs_[...]

    half = pl.BlockSpec((RB, D), lambda i: (i, 0))
    wspec = pl.BlockSpec((D, HD), lambda i: (0, 0))
    bspec = pl.BlockSpec((1, HD), lambda i: (0, 0))
    big = pl.BlockSpec((2, RB, 128), lambda i: (0, i, 0))
    return pl.pallas_call(
        body,
        grid=(5,),
        in_specs=[half, half, wspec, bspec, wspec, bspec, wspec, bspec,
                  wspec, bspec],
        out_specs=[big, big, big, pl.BlockSpec((RB, HD), lambda i: (i, 0))],
        out_shape=[jax.ShapeDtypeStruct((2, N, 128), jnp.float32)] * 3
        + [jax.ShapeDtypeStruct((N, HD), jnp.float32)],
    )(t, llc, Wq, bq, Wk, bk, Wv, bv, Ws, bs)


def _tc_cat(att, s, W, b):
    def body(a0, a1, s_ref, w_ref, b_ref, o_ref):
        cols = []
        for a in (a0[...], a1[...]):
            for j in range(8):
                den = a[:, 128 + j:129 + j] + 1e-16
                cols.append(a[:, j * 16:(j + 1) * 16] / den)
        tf = jnp.concatenate(cols, axis=1) + s_ref[...]
        o_ref[...] = (
            jnp.dot(tf, w_ref[...], preferred_element_type=jnp.float32)
            + b_ref[...]
        )

    return pl.pallas_call(
        body,
        grid=(5,),
        in_specs=[
            pl.BlockSpec((RB, 144), lambda i: (i, 0)),
            pl.BlockSpec((RB, 144), lambda i: (i + 5, 0)),
            pl.BlockSpec((RB, HD), lambda i: (i, 0)),
            pl.BlockSpec((HD, D), lambda i: (0, 0)),
            pl.BlockSpec((1, D), lambda i: (0, 0)),
        ],
        out_specs=pl.BlockSpec((RB, D), lambda i: (i, 0)),
        out_shape=jax.ShapeDtypeStruct((N, D), jnp.float32),
    )(att, att, s, W, b)


def _tc_ln(t_pre, g, b):
    def body(t_ref, g_ref, b_ref, o_ref):
        t = t_ref[...]
        m = jnp.mean(t)
        v = jnp.mean((t - m) ** 2)
        o_ref[...] = (t - m) / jnp.sqrt(v + 1e-5) * g_ref[...] + b_ref[...]

    return pl.pallas_call(
        body,
        out_shape=jax.ShapeDtypeStruct((N, D), jnp.float32),
    )(t_pre, g, b)


def _tc_lnff(t2_pre, g, b, Wl, Wr, bl):
    def body(t_ref, g_ref, b_ref, wl, wr, bl_, oz, ozr):
        t = t_ref[...]
        m = jnp.mean(t)
        v = jnp.mean((t - m) ** 2)
        t2 = (t - m) / jnp.sqrt(v + 1e-5) * g_ref[...] + b_ref[...]
        oz[...] = jnp.dot(t2, wl[...], preferred_element_type=jnp.float32)
        ozr[...] = (
            jnp.dot(t2, wr[...], preferred_element_type=jnp.float32) + bl_[...]
        )

    return pl.pallas_call(
        body,
        out_shape=[jax.ShapeDtypeStruct((N, D), jnp.float32)] * 2,
    )(t2_pre, g, b, Wl, Wr, bl)


def _tc_final(aggz, aggc, zr, g, b):
    def body(z0, z1, c0, c1, zr_ref, g_ref, b_ref, o_ref):
        cnt = jnp.maximum(c0[:, 0:1] + c1[:, 0:1], 1.0)
        t = (z0[...] + z1[...]) / cnt + zr_ref[...]
        m = jnp.mean(t)
        v = jnp.mean((t - m) ** 2)
        o_ref[...] = (t - m) / jnp.sqrt(v + 1e-5) * g_ref[...] + b_ref[...]

    return pl.pallas_call(
        body,
        out_shape=jax.ShapeDtypeStruct((N, D), jnp.float32),
    )(aggz[:N], aggz[N:], aggc[:N], aggc[N:], zr, g, b)


def _sc_sage(y, src, dst, with_count):
    mesh = plsc.VectorSubcoreMesh(core_axis_name="c", subcore_axis_name="s", num_cores=NC, num_subcores=NS)
    nout = 2 if with_count else 1
    scratch = [
        pltpu.VMEM((_CS,), jnp.int32),
        pltpu.VMEM((_CS,), jnp.int32),
        pltpu.VMEM((_CS, D), jnp.float32),
        pltpu.VMEM((_DC, D), jnp.float32),
        pltpu.VMEM_SHARED((N, D), jnp.float32),
    ]
    if with_count:
        scratch += [
            pltpu.VMEM((_CS, D), jnp.float32),
            pltpu.VMEM_SHARED((N, D), jnp.float32),
        ]

    @functools.partial(
        pl.kernel,
        out_type=[jax.ShapeDtypeStruct((NC * N, D), jnp.float32)] * nout,
        mesh=mesh,
        scratch_types=scratch,
        compiler_params=pltpu.CompilerParams(use_tc_tiling_on_sc=False, needs_layout_passes=False),
    )
    def kfn(y_hbm, src_hbm, dst_hbm, *rest):
        if with_count:
            outy, outc, si, di, yb, db, accy, cb, accc = rest
        else:
            outy, si, di, yb, db, accy = rest
            outc = cb = accc = None
        c = lax.axis_index("c")
        s = lax.axis_index("s")
        iot = lax.iota(jnp.int32, D)
        st = s * 624 + jnp.minimum(s, 2) * 8

        def zrow(i, _):
            db[i, pl.ds(0, D)] = jnp.zeros((D,), jnp.float32)
            return 0

        lax.fori_loop(0, _DC, zrow, 0)

        def zcp(g, _):
            r0 = st + g * _DC
            pltpu.sync_copy(db, accy.at[pl.ds(r0, _DC)])
            if with_count:
                pltpu.sync_copy(db, accc.at[pl.ds(r0, _DC)])
            return 0

        lax.fori_loop(0, 3, zcp, 0)

        @pl.when(s < 2)
        def _():
            r0 = st + 3 * _DC
            pltpu.sync_copy(db.at[pl.ds(0, 8)], accy.at[pl.ds(r0, 8)])
            if with_count:
                pltpu.sync_copy(db.at[pl.ds(0, 8)], accc.at[pl.ds(r0, 8)])

        if with_count:
            onec = jnp.where(iot == 0, 1.0, 0.0).astype(jnp.float32)

            def prep(e, _):
                cb[e, pl.ds(0, D)] = onec
                return 0

            lax.fori_loop(0, _CS, prep, 0)
        plsc.subcore_barrier()

        wid = c * NS + s

        def chunk(i, _):
            base = wid * (E // (NC * NS)) + i * _CS
            pltpu.sync_copy(src_hbm.at[pl.ds(base, _CS)], si)
            pltpu.sync_copy(dst_hbm.at[pl.ds(base, _CS)], di)
            pltpu.sync_copy(y_hbm.at[si], yb)
            pltpu.sync_copy(yb, accy.at[di], add=True)
            if with_count:
                pltpu.sync_copy(cb, accc.at[di], add=True)
            return 0

        lax.fori_loop(0, _NCHS, chunk, 0)
        plsc.subcore_barrier()

        def drain(g, _):
            r0 = st + g * _DC
            pltpu.sync_copy(accy.at[pl.ds(r0, _DC)], outy.at[pl.ds(c * N + r0, _DC)])
            if with_count:
                pltpu.sync_copy(accc.at[pl.ds(r0, _DC)], outc.at[pl.ds(c * N + r0, _DC)])
            return 0

        lax.fori_loop(0, 3, drain, 0)

        @pl.when(s < 2)
        def _():
            r0 = st + 3 * _DC
            pltpu.sync_copy(accy.at[pl.ds(r0, 8)], outy.at[pl.ds(c * N + r0, 8)])
            if with_count:
                pltpu.sync_copy(accc.at[pl.ds(r0, 8)], outc.at[pl.ds(c * N + r0, 8)])

    return kfn(y, src, dst)


def _sc_attn(q, k, v, ei):
    mesh = plsc.VectorSubcoreMesh(core_axis_name="c", subcore_axis_name="s", num_cores=NC, num_subcores=NS)

    nset = 2
    per_set = [
        pltpu.VMEM((2, _CA), jnp.int32),
        pltpu.VMEM((_CA,), jnp.int32),
        pltpu.VMEM((_CA,), jnp.int32),
        pltpu.VMEM((_CA,), jnp.int32),
        pltpu.VMEM((_CA, 128), jnp.float32),
        pltpu.VMEM((_CA, 128), jnp.float32),
        pltpu.VMEM((_CA, 128), jnp.float32),
        pltpu.VMEM((_CA, 144), jnp.float32),
        pltpu.SemaphoreType.DMA,
        pltpu.SemaphoreType.DMA,
    ]

    @functools.partial(
        pl.kernel,
        out_type=jax.ShapeDtypeStruct((NC * N, 144), jnp.float32),
        mesh=mesh,
        scratch_types=per_set * nset + [
            pltpu.VMEM_SHARED((N + 16, 144), jnp.float32),
        ],
        compiler_params=pltpu.CompilerParams(use_tc_tiling_on_sc=False, needs_layout_passes=False),
    )
    def kfn(q_hbm, k_hbm, v_hbm, ei_hbm, out_hbm, *scr):
        sets = [scr[i * 10:(i + 1) * 10] for i in range(nset)]
        acc = scr[nset * 10]
        c = lax.axis_index("c")
        s = lax.axis_index("s")
        cN = c * N
        iot = lax.iota(jnp.int32, 16)
        st = s * 624 + jnp.minimum(s, 2) * 8
        stg0 = sets[0][7]

        def zrow(i, _):
            for j in range(9):
                stg0[i, pl.ds(j * 16, 16)] = jnp.zeros((16,), jnp.float32)
            return 0

        lax.fori_loop(0, _CA, zrow, 0)

        def zcp(g, _):
            pltpu.sync_copy(stg0, acc.at[pl.ds(st + g * _CA, _CA)])
            return 0

        lax.fori_loop(0, 19, zcp, 0)
        pltpu.sync_copy(stg0.at[pl.ds(0, 16)], acc.at[pl.ds(st + 608, 16)])

        @pl.when(s < 2)
        def _():
            pltpu.sync_copy(stg0.at[pl.ds(0, 8)], acc.at[pl.ds(st + 624, 8)])

        @pl.when(s == 0)
        def _():
            pltpu.sync_copy(stg0.at[pl.ds(0, 16)], acc.at[pl.ds(N, 16)])

        plsc.subcore_barrier()

        cbase = s * _NCHA

        def load_and_fire(i, st_):
            eb, gsk, gq = st_[0], st_[1], st_[2]
            qb, kb, vb, gsem = st_[4], st_[5], st_[6], st_[8]
            pltpu.sync_copy(ei_hbm.at[cbase + i], eb)
            for j in range(_CA // 16):
                sl = pl.ds(j * 16, 16)
                gsk[sl] = eb[0, sl] + cN
                gq[sl] = jnp.minimum(eb[1, sl], N - 1) + cN
            pltpu.async_copy(q_hbm.at[gq], qb, gsem)
            pltpu.async_copy(k_hbm.at[gsk], kb, gsem)
            pltpu.async_copy(v_hbm.at[gsk], vb, gsem)

        def run_chunk(i, g, st_, st_next):
            eb, gsk, gq, dsc, qb, kb, vb, stg, gsem, ssem = st_
            pltpu.make_async_copy(q_hbm.at[gq], qb, gsem).wait()
            pltpu.make_async_copy(k_hbm.at[gsk], kb, gsem).wait()
            pltpu.make_async_copy(v_hbm.at[gsk], vb, gsem).wait()

            @pl.when(i + 1 < _NCHA)
            def _():
                load_and_fire(i + 1, st_next)

            @pl.when(g > 0)
            def _():
                pltpu.make_async_copy(stg, acc.at[dsc], ssem).wait()

            for j in range(_CA // 16):
                sl = pl.ds(j * 16, 16)
                dsc[sl] = eb[1, sl]

            @plsc.parallel_loop(0, _CA, step=1, unroll=4)
            def edge(e):
                exv = jnp.zeros((16,), jnp.float32)
                for j in range(8):
                    sl = pl.ds(j * 16, 16)
                    lg = jnp.sum(qb[e, sl] * kb[e, sl])
                    ev = jnp.exp(jnp.broadcast_to(lg, (16,)))
                    stg[e, sl] = vb[e, sl] * ev
                    exv = jnp.where(iot == j, ev, exv)
                stg[e, pl.ds(128, 16)] = exv

            pltpu.async_copy(stg, acc.at[dsc], ssem, add=True)

        load_and_fire(0, sets[0])

        def pair(g, _):
            run_chunk(2 * g, g, sets[0], sets[1])
            run_chunk(2 * g + 1, g, sets[1], sets[0])
            return 0

        lax.fori_loop(0, _NCHA // 2, pair, 0)
        for b in range(nset):
            dsc, stg, ssem = sets[b][3], sets[b][7], sets[b][9]
            pltpu.make_async_copy(stg, acc.at[dsc], ssem).wait()
        plsc.subcore_barrier()

        def drain(g, _):
            r0 = st + g * _DC
            pltpu.sync_copy(acc.at[pl.ds(r0, _DC)], out_hbm.at[pl.ds(cN + r0, _DC)])
            return 0

        lax.fori_loop(0, 3, drain, 0)

        @pl.when(s < 2)
        def _():
            r0 = st + 3 * _DC
            pltpu.sync_copy(acc.at[pl.ds(r0, 8)], out_hbm.at[pl.ds(cN + r0, 8)])

    return kfn(q, k, v, ei)


def kernel(x, llc_x, edge_index, params):
    p = params
    src = edge_index[0].astype(jnp.int32)
    dst = edge_index[1].astype(jnp.int32)
    pad = _EP - E
    src_p = jnp.concatenate([src, jnp.zeros((pad,), jnp.int32)])
    dst_p = jnp.concatenate([dst, jnp.full((pad,), N, jnp.int32)])
    ei = jnp.stack([src_p.reshape(-1, _CA), dst_p.reshape(-1, _CA)], axis=1)

    def b2(a):
        return a.reshape(1, -1)

    y, xr = _tc_embed(x, p['emb_Wl'], p['emb_Wr'])
    aggy, aggc = _sc_sage(y, src, dst, with_count=True)
    a1 = p['a1']
    q1, k1, v1, s1 = _tc_qkv1(
        aggy, aggc, xr, b2(p['emb_bl']),
        a1['Wq'], b2(a1['bq']), a1['Wk'], b2(a1['bk']),
        a1['Wv'], b2(a1['bv']), a1['Ws'], b2(a1['bs']))
    att1 = _sc_attn(q1.reshape(NC * N, 128), k1.reshape(NC * N, 128),
                    v1.reshape(NC * N, 128), ei)
    t_pre = _tc_cat(att1, s1, p['cat1_W'], b2(p['cat1_b']))
    t = _tc_ln(t_pre, b2(p['ln1_g']), b2(p['ln1_b']))

    a2 = p['a2']
    q2, k2, v2, s2 = _tc_qkv2(
        t, llc_x,
        a2['Wq'], b2(a2['bq']), a2['Wk'], b2(a2['bk']),
        a2['Wv'], b2(a2['bv']), a2['Ws'], b2(a2['bs']))
    att2 = _sc_attn(q2.reshape(NC * N, 128), k2.reshape(NC * N, 128),
                    v2.reshape(NC * N, 128), ei)
    t2_pre = _tc_cat(att2, s2, p['cat2_W'], b2(p['cat2_b']))
    z, zr = _tc_lnff(t2_pre, b2(p['ln2_g']), b2(p['ln2_b']),
                     p['ff_Wl'], p['ff_Wr'], b2(p['ff_bl']))

    (aggz,) = _sc_sage(z, src, dst, with_count=False)
    return _tc_final(aggz, aggc, zr, b2(p['ln3_g']), b2(p['ln3_b']))


# speedup vs baseline: 28.9547x; 1.0478x over previous
import functools

import jax
import jax.numpy as jnp
from jax import lax
from jax.experimental import pallas as pl
from jax.experimental.pallas import tpu as pltpu
from jax.experimental.pallas import tpu_sc as plsc

N = 10000
E = 160000
D_IN = 128
D = 16
H = 16
HD = 256
NC = 2
NS = 16
LPT = N // NS
RB = N // 5

_CA = 32
_EP = 160768
_NCHA = (_EP // NS) // _CA
_CS = 40
_DC = 208
_NCHS = (E // (NC * NS)) // _CS


def _tc_embed(x, Wl, Wr):
    def body(x_ref, wl_ref, wr_ref, y_ref, xr_ref):
        xb = x_ref[...]
        y_ref[...] = jnp.dot(xb, wl_ref[...], preferred_element_type=jnp.float32)
        xr_ref[...] = jnp.dot(xb, wr_ref[...], preferred_element_type=jnp.float32)

    return pl.pallas_call(
        body,
        grid=(5,),
        in_specs=[
            pl.BlockSpec((RB, D_IN), lambda i: (i, 0)),
            pl.BlockSpec((D_IN, D), lambda i: (0, 0)),
            pl.BlockSpec((D_IN, D), lambda i: (0, 0)),
        ],
        out_specs=[
            pl.BlockSpec((RB, D), lambda i: (i, 0)),
            pl.BlockSpec((RB, D), lambda i: (i, 0)),
        ],
        out_shape=[jax.ShapeDtypeStruct((N, D), jnp.float32)] * 2,
    )(x, Wl, Wr)


def _tc_qkv1(aggy, aggc, xr, bl, Wq, bq, Wk, bk, Wv, bv, Ws, bs):
    def body(a0, a1, c0, c1, xr_ref, bl_ref, wq, bq_, wk, bk_, wv, bv_,
             ws, bs_, oq, ok, ov, os_):
        cnt = jnp.maximum(c0[:, 0:1] + c1[:, 0:1], 1.0)
        h = (a0[...] + a1[...]) / cnt + bl_ref[...] + xr_ref[...]
        q = (jnp.dot(h, wq[...], preferred_element_type=jnp.float32)
             + bq_[...]) * 0.25
        k = jnp.dot(h, wk[...], preferred_element_type=jnp.float32) + bk_[...]
        v = jnp.dot(h, wv[...], preferred_element_type=jnp.float32) + bv_[...]
        oq[0] = q[:, :128]
        oq[1] = q[:, 128:]
        ok[0] = k[:, :128]
        ok[1] = k[:, 128:]
        ov[0] = v[:, :128]
        ov[1] = v[:, 128:]
        os_[...] = jnp.dot(h, ws[...], preferred_element_type=jnp.float32) + bs_[...]

    half = pl.BlockSpec((RB, D), lambda i: (i, 0))
    wspec = pl.BlockSpec((D, HD), lambda i: (0, 0))
    bspec = pl.BlockSpec((1, HD), lambda i: (0, 0))
    big = pl.BlockSpec((2, RB, 128), lambda i: (0, i, 0))
    return pl.pallas_call(
        body,
        grid=(5,),
        in_specs=[half, half, half, half, half,
                  pl.BlockSpec((1, D), lambda i: (0, 0)),
                  wspec, bspec, wspec, bspec, wspec, bspec, wspec, bspec],
        out_specs=[big, big, big, pl.BlockSpec((RB, HD), lambda i: (i, 0))],
        out_shape=[jax.ShapeDtypeStruct((2, N, 128), jnp.float32)] * 3
        + [jax.ShapeDtypeStruct((N, HD), jnp.float32)],
    )(aggy[:N], aggy[N:], aggc[:N], aggc[N:], xr, bl,
      Wq, bq, Wk, bk, Wv, bv, Ws, bs)


def _tc_qkv2(t, llc, Wq, bq, Wk, bk, Wv, bv, Ws, bs):
    def body(t_ref, l_ref, wq, bq_, wk, bk_, wv, bv_, ws, bs_,
             oq, ok, ov, os_):
        tb = t_ref[...]
        lb = l_ref[...]
        q = (jnp.dot(lb, wq[...], preferred_element_type=jnp.float32)
             + bq_[...]) * 0.25
        k = jnp.dot(tb, wk[...], preferred_element_type=jnp.float32) + bk_[...]
        v = jnp.dot(tb, wv[...], preferred_element_type=jnp.float32) + bv_[...]
        oq[0] = q[:, :128]
        oq[1] = q[:, 128:]
        ok[0] = k[:, :128]
        ok[1] = k[:, 128:]
        ov[0] = v[:, :128]
        ov[1] = v[:, 128:]
        os_[...] = jnp.dot(lb, ws[...], preferred_element_type=jnp.float32) + bs_[...]

    half = pl.BlockSpec((RB, D), lambda i: (i, 0))
    wspec = pl.BlockSpec((D, HD), lambda i: (0, 0))
    bspec = pl.BlockSpec((1, HD), lambda i: (0, 0))
    big = pl.BlockSpec((2, RB, 128), lambda i: (0, i, 0))
    return pl.pallas_call(
        body,
        grid=(5,),
        in_specs=[half, half, wspec, bspec, wspec, bspec, wspec, bspec,
                  wspec, bspec],
        out_specs=[big, big, big, pl.BlockSpec((RB, HD), lambda i: (i, 0))],
        out_shape=[jax.ShapeDtypeStruct((2, N, 128), jnp.float32)] * 3
        + [jax.ShapeDtypeStruct((N, HD), jnp.float32)],
    )(t, llc, Wq, bq, Wk, bk, Wv, bv, Ws, bs)


def _tc_cat(att, s, W, b):
    def body(a0, a1, s_ref, w_ref, b_ref, o_ref):
        cols = []
        for a in (a0[...], a1[...]):
            for j in range(8):
                den = a[:, 128 + j:129 + j] + 1e-16
                cols.append(a[:, j * 16:(j + 1) * 16] / den)
        tf = jnp.concatenate(cols, axis=1) + s_ref[...]
        o_ref[...] = (
            jnp.dot(tf, w_ref[...], preferred_element_type=jnp.float32)
            + b_ref[...]
        )

    return pl.pallas_call(
        body,
        grid=(5,),
        in_specs=[
            pl.BlockSpec((RB, 144), lambda i: (i, 0)),
            pl.BlockSpec((RB, 144), lambda i: (i + 5, 0)),
            pl.BlockSpec((RB, HD), lambda i: (i, 0)),
            pl.BlockSpec((HD, D), lambda i: (0, 0)),
            pl.BlockSpec((1, D), lambda i: (0, 0)),
        ],
        out_specs=pl.BlockSpec((RB, D), lambda i: (i, 0)),
        out_shape=jax.ShapeDtypeStruct((N, D), jnp.float32),
    )(att, att, s, W, b)


def _tc_ln(t_pre, g, b):
    def body(t_ref, g_ref, b_ref, o_ref):
        t = t_ref[...]
        m = jnp.mean(t)
        v = jnp.mean((t - m) ** 2)
        o_ref[...] = (t - m) / jnp.sqrt(v + 1e-5) * g_ref[...] + b_ref[...]

    return pl.pallas_call(
        body,
        out_shape=jax.ShapeDtypeStruct((N, D), jnp.float32),
    )(t_pre, g, b)


def _tc_lnff(t2_pre, g, b, Wl, Wr, bl):
    def body(t_ref, g_ref, b_ref, wl, wr, bl_, oz, ozr):
        t = t_ref[...]
        m = jnp.mean(t)
        v = jnp.mean((t - m) ** 2)
        t2 = (t - m) / jnp.sqrt(v + 1e-5) * g_ref[...] + b_ref[...]
        oz[...] = jnp.dot(t2, wl[...], preferred_element_type=jnp.float32)
        ozr[...] = (
            jnp.dot(t2, wr[...], preferred_element_type=jnp.float32) + bl_[...]
        )

    return pl.pallas_call(
        body,
        out_shape=[jax.ShapeDtypeStruct((N, D), jnp.float32)] * 2,
    )(t2_pre, g, b, Wl, Wr, bl)


def _tc_final(aggz, aggc, zr, g, b):
    def body(z0, z1, c0, c1, zr_ref, g_ref, b_ref, o_ref):
        cnt = jnp.maximum(c0[:, 0:1] + c1[:, 0:1], 1.0)
        t = (z0[...] + z1[...]) / cnt + zr_ref[...]
        m = jnp.mean(t)
        v = jnp.mean((t - m) ** 2)
        o_ref[...] = (t - m) / jnp.sqrt(v + 1e-5) * g_ref[...] + b_ref[...]

    return pl.pallas_call(
        body,
        out_shape=jax.ShapeDtypeStruct((N, D), jnp.float32),
    )(aggz[:N], aggz[N:], aggc[:N], aggc[N:], zr, g, b)


def _sc_sage(y, src, dst, with_count):
    mesh = plsc.VectorSubcoreMesh(core_axis_name="c", subcore_axis_name="s", num_cores=NC, num_subcores=NS)
    nout = 2 if with_count else 1
    scratch = [
        pltpu.VMEM((_CS,), jnp.int32),
        pltpu.VMEM((_CS,), jnp.int32),
        pltpu.VMEM((_CS, D), jnp.float32),
        pltpu.VMEM((_DC, D), jnp.float32),
        pltpu.VMEM_SHARED((N, D), jnp.float32),
    ]
    if with_count:
        scratch += [
            pltpu.VMEM((_CS, D), jnp.float32),
            pltpu.VMEM_SHARED((N, D), jnp.float32),
        ]

    @functools.partial(
        pl.kernel,
        out_type=[jax.ShapeDtypeStruct((NC * N, D), jnp.float32)] * nout,
        mesh=mesh,
        scratch_types=scratch,
        compiler_params=pltpu.CompilerParams(use_tc_tiling_on_sc=False, needs_layout_passes=False),
    )
    def kfn(y_hbm, src_hbm, dst_hbm, *rest):
        if with_count:
            outy, outc, si, di, yb, db, accy, cb, accc = rest
        else:
            outy, si, di, yb, db, accy = rest
            outc = cb = accc = None
        c = lax.axis_index("c")
        s = lax.axis_index("s")
        iot = lax.iota(jnp.int32, D)
        st = s * 624 + jnp.minimum(s, 2) * 8

        def zrow(i, _):
            db[i, pl.ds(0, D)] = jnp.zeros((D,), jnp.float32)
            return 0

        lax.fori_loop(0, _DC, zrow, 0)

        def zcp(g, _):
            r0 = st + g * _DC
            pltpu.sync_copy(db, accy.at[pl.ds(r0, _DC)])
            if with_count:
                pltpu.sync_copy(db, accc.at[pl.ds(r0, _DC)])
            return 0

        lax.fori_loop(0, 3, zcp, 0)

        @pl.when(s < 2)
        def _():
            r0 = st + 3 * _DC
            pltpu.sync_copy(db.at[pl.ds(0, 8)], accy.at[pl.ds(r0, 8)])
            if with_count:
                pltpu.sync_copy(db.at[pl.ds(0, 8)], accc.at[pl.ds(r0, 8)])

        if with_count:
            onec = jnp.where(iot == 0, 1.0, 0.0).astype(jnp.float32)

            def prep(e, _):
                cb[e, pl.ds(0, D)] = onec
                return 0

            lax.fori_loop(0, _CS, prep, 0)
        plsc.subcore_barrier()

        wid = c * NS + s

        def chunk(i, _):
            base = wid * (E // (NC * NS)) + i * _CS
            pltpu.sync_copy(src_hbm.at[pl.ds(base, _CS)], si)
            pltpu.sync_copy(dst_hbm.at[pl.ds(base, _CS)], di)
            pltpu.sync_copy(y_hbm.at[si], yb)
            pltpu.sync_copy(yb, accy.at[di], add=True)
            if with_count:
                pltpu.sync_copy(cb, accc.at[di], add=True)
            return 0

        lax.fori_loop(0, _NCHS, chunk, 0)
        plsc.subcore_barrier()

        def drain(g, _):
            r0 = st + g * _DC
            pltpu.sync_copy(accy.at[pl.ds(r0, _DC)], outy.at[pl.ds(c * N + r0, _DC)])
            if with_count:
                pltpu.sync_copy(accc.at[pl.ds(r0, _DC)], outc.at[pl.ds(c * N + r0, _DC)])
            return 0

        lax.fori_loop(0, 3, drain, 0)

        @pl.when(s < 2)
        def _():
            r0 = st + 3 * _DC
            pltpu.sync_copy(accy.at[pl.ds(r0, 8)], outy.at[pl.ds(c * N + r0, 8)])
            if with_count:
                pltpu.sync_copy(accc.at[pl.ds(r0, 8)], outc.at[pl.ds(c * N + r0, 8)])

    return kfn(y, src, dst)


def _sc_attn(q, k, v, ei):
    mesh = plsc.VectorSubcoreMesh(core_axis_name="c", subcore_axis_name="s", num_cores=NC, num_subcores=NS)

    nset = 2
    per_set = [
        pltpu.VMEM((2, _CA), jnp.int32),
        pltpu.VMEM((_CA,), jnp.int32),
        pltpu.VMEM((_CA,), jnp.int32),
        pltpu.VMEM((_CA,), jnp.int32),
        pltpu.VMEM((_CA, 128), jnp.float32),
        pltpu.VMEM((_CA, 128), jnp.float32),
        pltpu.VMEM((_CA, 128), jnp.float32),
        pltpu.VMEM((_CA, 144), jnp.float32),
        pltpu.SemaphoreType.DMA,
        pltpu.SemaphoreType.DMA,
    ]

    @functools.partial(
        pl.kernel,
        out_type=jax.ShapeDtypeStruct((NC * N, 144), jnp.float32),
        mesh=mesh,
        scratch_types=per_set * nset + [
            pltpu.VMEM_SHARED((N + 16, 144), jnp.float32),
        ],
        compiler_params=pltpu.CompilerParams(use_tc_tiling_on_sc=False, needs_layout_passes=False),
    )
    def kfn(q_hbm, k_hbm, v_hbm, ei_hbm, out_hbm, *scr):
        sets = [scr[i * 10:(i + 1) * 10] for i in range(nset)]
        acc = scr[nset * 10]
        c = lax.axis_index("c")
        s = lax.axis_index("s")
        cN = c * N
        iot = lax.iota(jnp.int32, 16)
        st = s * 624 + jnp.minimum(s, 2) * 8
        stg0 = sets[0][7]

        def zrow(i, _):
            for j in range(9):
                stg0[i, pl.ds(j * 16, 16)] = jnp.zeros((16,), jnp.float32)
            return 0

        lax.fori_loop(0, _CA, zrow, 0)

        def zcp(g, _):
            pltpu.sync_copy(stg0, acc.at[pl.ds(st + g * _CA, _CA)])
            return 0

        lax.fori_loop(0, 19, zcp, 0)
        pltpu.sync_copy(stg0.at[pl.ds(0, 16)], acc.at[pl.ds(st + 608, 16)])

        @pl.when(s < 2)
        def _():
            pltpu.sync_copy(stg0.at[pl.ds(0, 8)], acc.at[pl.ds(st + 624, 8)])

        @pl.when(s == 0)
        def _():
            pltpu.sync_copy(stg0.at[pl.ds(0, 16)], acc.at[pl.ds(N, 16)])

        plsc.subcore_barrier()

        cbase = s * _NCHA

        def load_and_fire(i, st_):
            eb, gsk, gq = st_[0], st_[1], st_[2]
            qb, kb, vb, gsem = st_[4], st_[5], st_[6], st_[8]
            pltpu.sync_copy(ei_hbm.at[cbase + i], eb)
            for j in range(_CA // 16):
                sl = pl.ds(j * 16, 16)
                gsk[sl] = eb[0, sl] + cN
                gq[sl] = jnp.minimum(eb[1, sl], N - 1) + cN
            pltpu.async_copy(q_hbm.at[gq], qb, gsem)
            pltpu.async_copy(k_hbm.at[gsk], kb, gsem)
            pltpu.async_copy(v_hbm.at[gsk], vb, gsem)

        def run_chunk(i, g, st_, st_next):
            eb, gsk, gq, dsc, qb, kb, vb, stg, gsem, ssem = st_
            pltpu.make_async_copy(q_hbm.at[gq], qb, gsem).wait()
            pltpu.make_async_copy(k_hbm.at[gsk], kb, gsem).wait()
            pltpu.make_async_copy(v_hbm.at[gsk], vb, gsem).wait()

            @pl.when(i + 1 < _NCHA)
            def _():
                load_and_fire(i + 1, st_next)

            @pl.when(g > 0)
            def _():
                pltpu.make_async_copy(stg, acc.at[dsc], ssem).wait()

            for j in range(_CA // 16):
                sl = pl.ds(j * 16, 16)
                dsc[sl] = eb[1, sl]

            @plsc.parallel_loop(0, _CA, step=1, unroll=4)
            def edge(e):
                lgv = jnp.zeros((16,), jnp.float32)
                for j in range(8):
                    sl = pl.ds(j * 16, 16)
                    lg = jnp.sum(qb[e, sl] * kb[e, sl])
                    lgv = jnp.where(iot == j, lg, lgv)
                exv = jnp.exp(lgv)
                for j in range(8):
                    sl = pl.ds(j * 16, 16)
                    stg[e, sl] = vb[e, sl] * exv[j]
                stg[e, pl.ds(128, 16)] = exv

            pltpu.async_copy(stg, acc.at[dsc], ssem, add=True)

        load_and_fire(0, sets[0])

        def pair(g, _):
            run_chunk(2 * g, g, sets[0], sets[1])
            run_chunk(2 * g + 1, g, sets[1], sets[0])
            return 0

        lax.fori_loop(0, _NCHA // 2, pair, 0)
        for b in range(nset):
            dsc, stg, ssem = sets[b][3], sets[b][7], sets[b][9]
            pltpu.make_async_copy(stg, acc.at[dsc], ssem).wait()
        plsc.subcore_barrier()

        def drain(g, _):
            r0 = st + g * _DC
            pltpu.sync_copy(acc.at[pl.ds(r0, _DC)], out_hbm.at[pl.ds(cN + r0, _DC)])
            return 0

        lax.fori_loop(0, 3, drain, 0)

        @pl.when(s < 2)
        def _():
            r0 = st + 3 * _DC
            pltpu.sync_copy(acc.at[pl.ds(r0, 8)], out_hbm.at[pl.ds(cN + r0, 8)])

    return kfn(q, k, v, ei)


def kernel(x, llc_x, edge_index, params):
    p = params
    src = edge_index[0].astype(jnp.int32)
    dst = edge_index[1].astype(jnp.int32)
    pad = _EP - E
    src_p = jnp.concatenate([src, jnp.zeros((pad,), jnp.int32)])
    dst_p = jnp.concatenate([dst, jnp.full((pad,), N, jnp.int32)])
    ei = jnp.stack([src_p.reshape(-1, _CA), dst_p.reshape(-1, _CA)], axis=1)

    def b2(a):
        return a.reshape(1, -1)

    y, xr = _tc_embed(x, p['emb_Wl'], p['emb_Wr'])
    aggy, aggc = _sc_sage(y, src, dst, with_count=True)
    a1 = p['a1']
    q1, k1, v1, s1 = _tc_qkv1(
        aggy, aggc, xr, b2(p['emb_bl']),
        a1['Wq'], b2(a1['bq']), a1['Wk'], b2(a1['bk']),
        a1['Wv'], b2(a1['bv']), a1['Ws'], b2(a1['bs']))
    att1 = _sc_attn(q1.reshape(NC * N, 128), k1.reshape(NC * N, 128),
                    v1.reshape(NC * N, 128), ei)
    t_pre = _tc_cat(att1, s1, p['cat1_W'], b2(p['cat1_b']))
    t = _tc_ln(t_pre, b2(p['ln1_g']), b2(p['ln1_b']))

    a2 = p['a2']
    q2, k2, v2, s2 = _tc_qkv2(
        t, llc_x,
        a2['Wq'], b2(a2['bq']), a2['Wk'], b2(a2['bk']),
        a2['Wv'], b2(a2['bv']), a2['Ws'], b2(a2['bs']))
    att2 = _sc_attn(q2.reshape(NC * N, 128), k2.reshape(NC * N, 128),
                    v2.reshape(NC * N, 128), ei)
    t2_pre = _tc_cat(att2, s2, p['cat2_W'], b2(p['cat2_b']))
    z, zr = _tc_lnff(t2_pre, b2(p['ln2_g']), b2(p['ln2_b']),
                     p['ff_Wl'], p['ff_Wr'], b2(p['ff_bl']))

    (aggz,) = _sc_sage(z, src, dst, with_count=False)
    return _tc_final(aggz, aggc, zr, b2(p['ln3_g']), b2(p['ln3_b']))


# speedup vs baseline: 28.9675x; 1.0004x over previous
import functools

import jax
import jax.numpy as jnp
from jax import lax
from jax.experimental import pallas as pl
from jax.experimental.pallas import tpu as pltpu
from jax.experimental.pallas import tpu_sc as plsc

N = 10000
E = 160000
D_IN = 128
D = 16
H = 16
HD = 256
NC = 2
NS = 16
LPT = N // NS
RB = N // 5

_CA = 32
_EP = 160768
_NCHA = (_EP // NS) // _CA
_CS = 40
_DC = 208
_NCHS = (E // (NC * NS)) // _CS


def _tc_embed(x, Wl, Wr):
    def body(x_ref, wl_ref, wr_ref, y_ref, xr_ref):
        xb = x_ref[...]
        y_ref[...] = jnp.dot(xb, wl_ref[...], preferred_element_type=jnp.float32)
        xr_ref[...] = jnp.dot(xb, wr_ref[...], preferred_element_type=jnp.float32)

    return pl.pallas_call(
        body,
        grid=(5,),
        in_specs=[
            pl.BlockSpec((RB, D_IN), lambda i: (i, 0)),
            pl.BlockSpec((D_IN, D), lambda i: (0, 0)),
            pl.BlockSpec((D_IN, D), lambda i: (0, 0)),
        ],
        out_specs=[
            pl.BlockSpec((RB, D), lambda i: (i, 0)),
            pl.BlockSpec((RB, D), lambda i: (i, 0)),
        ],
        out_shape=[jax.ShapeDtypeStruct((N, D), jnp.float32)] * 2,
    )(x, Wl, Wr)


def _tc_qkv1(aggy, aggc, xr, bl, Wq, bq, Wk, bk, Wv, bv, Ws, bs):
    def body(a0, a1, c0, c1, xr_ref, bl_ref, wq, bq_, wk, bk_, wv, bv_,
             ws, bs_, oq, ok, ov, os_):
        cnt = jnp.maximum(c0[:, 0:1] + c1[:, 0:1], 1.0)
        h = (a0[...] + a1[...]) / cnt + bl_ref[...] + xr_ref[...]
        q = (jnp.dot(h, wq[...], preferred_element_type=jnp.float32)
             + bq_[...]) * 0.25
        k = jnp.dot(h, wk[...], preferred_element_type=jnp.float32) + bk_[...]
        v = jnp.dot(h, wv[...], preferred_element_type=jnp.float32) + bv_[...]
        oq[0] = q[:, :128]
        oq[1] = q[:, 128:]
        ok[0] = k[:, :128]
        ok[1] = k[:, 128:]
        ov[0] = v[:, :128]
        ov[1] = v[:, 128:]
        os_[...] = jnp.dot(h, ws[...], preferred_element_type=jnp.float32) + bs_[...]

    half = pl.BlockSpec((RB, D), lambda i: (i, 0))
    wspec = pl.BlockSpec((D, HD), lambda i: (0, 0))
    bspec = pl.BlockSpec((1, HD), lambda i: (0, 0))
    big = pl.BlockSpec((2, RB, 128), lambda i: (0, i, 0))
    return pl.pallas_call(
        body,
        grid=(5,),
        in_specs=[half, half, half, half, half,
                  pl.BlockSpec((1, D), lambda i: (0, 0)),
                  wspec, bspec, wspec, bspec, wspec, bspec, wspec, bspec],
        out_specs=[big, big, big, pl.BlockSpec((RB, HD), lambda i: (i, 0))],
        out_shape=[jax.ShapeDtypeStruct((2, N, 128), jnp.float32)] * 3
        + [jax.ShapeDtypeStruct((N, HD), jnp.float32)],
    )(aggy[:N], aggy[N:], aggc[:N], aggc[N:], xr, bl,
      Wq, bq, Wk, bk, Wv, bv, Ws, bs)


def _tc_qkv2(t, llc, Wq, bq, Wk, bk, Wv, bv, Ws, bs):
    def body(t_ref, l_ref, wq, bq_, wk, bk_, wv, bv_, ws, bs_,
             oq, ok, ov, os_):
        tb = t_ref[...]
        lb = l_ref[...]
        q = (jnp.dot(lb, wq[...], preferred_element_type=jnp.float32)
             + bq_[...]) * 0.25
        k = jnp.dot(tb, wk[...], preferred_element_type=jnp.float32) + bk_[...]
        v = jnp.dot(tb, wv[...], preferred_element_type=jnp.float32) + bv_[...]
        oq[0] = q[:, :128]
        oq[1] = q[:, 128:]
        ok[0] = k[:, :128]
        ok[1] = k[:, 128:]
        ov[0] = v[:, :128]
        ov[1] = v[:, 128:]
        os_[...] = jnp.dot(lb, ws[...], preferred_element_type=jnp.float32) + bs_[...]

    half = pl.BlockSpec((RB, D), lambda i: (i, 0))
    wspec = pl.BlockSpec((D, HD), lambda i: (0, 0))
    bspec = pl.BlockSpec((1, HD), lambda i: (0, 0))
    big = pl.BlockSpec((2, RB, 128), lambda i: (0, i, 0))
    return pl.pallas_call(
        body,
        grid=(5,),
        in_specs=[half, half, wspec, bspec, wspec, bspec, wspec, bspec,
                  wspec, bspec],
        out_specs=[big, big, big, pl.BlockSpec((RB, HD), lambda i: (i, 0))],
        out_shape=[jax.ShapeDtypeStruct((2, N, 128), jnp.float32)] * 3
        + [jax.ShapeDtypeStruct((N, HD), jnp.float32)],
    )(t, llc, Wq, bq, Wk, bk, Wv, bv, Ws, bs)


def _tc_cat(att, s, W, b):
    def body(a0, a1, s_ref, w_ref, b_ref, o_ref):
        cols = []
        for a in (a0[...], a1[...]):
            for j in range(8):
                den = a[:, 128 + j:129 + j] + 1e-16
                cols.append(a[:, j * 16:(j + 1) * 16] / den)
        tf = jnp.concatenate(cols, axis=1) + s_ref[...]
        o_ref[...] = (
            jnp.dot(tf, w_ref[...], preferred_element_type=jnp.float32)
            + b_ref[...]
        )

    return pl.pallas_call(
        body,
        grid=(5,),
        in_specs=[
            pl.BlockSpec((RB, 144), lambda i: (i, 0)),
            pl.BlockSpec((RB, 144), lambda i: (i + 5, 0)),
            pl.BlockSpec((RB, HD), lambda i: (i, 0)),
            pl.BlockSpec((HD, D), lambda i: (0, 0)),
            pl.BlockSpec((1, D), lambda i: (0, 0)),
        ],
        out_specs=pl.BlockSpec((RB, D), lambda i: (i, 0)),
        out_shape=jax.ShapeDtypeStruct((N, D), jnp.float32),
    )(att, att, s, W, b)


def _tc_ln(t_pre, g, b):
    def body(t_ref, g_ref, b_ref, o_ref):
        t = t_ref[...]
        m = jnp.mean(t)
        v = jnp.mean((t - m) ** 2)
        o_ref[...] = (t - m) / jnp.sqrt(v + 1e-5) * g_ref[...] + b_ref[...]

    return pl.pallas_call(
        body,
        out_shape=jax.ShapeDtypeStruct((N, D), jnp.float32),
    )(t_pre, g, b)


def _tc_lnff(t2_pre, g, b, Wl, Wr, bl):
    def body(t_ref, g_ref, b_ref, wl, wr, bl_, oz, ozr):
        t = t_ref[...]
        m = jnp.mean(t)
        v = jnp.mean((t - m) ** 2)
        t2 = (t - m) / jnp.sqrt(v + 1e-5) * g_ref[...] + b_ref[...]
        oz[...] = jnp.dot(t2, wl[...], preferred_element_type=jnp.float32)
        ozr[...] = (
            jnp.dot(t2, wr[...], preferred_element_type=jnp.float32) + bl_[...]
        )

    return pl.pallas_call(
        body,
        out_shape=[jax.ShapeDtypeStruct((N, D), jnp.float32)] * 2,
    )(t2_pre, g, b, Wl, Wr, bl)


def _tc_final(aggz, aggc, zr, g, b):
    def body(z0, z1, c0, c1, zr_ref, g_ref, b_ref, o_ref):
        cnt = jnp.maximum(c0[:, 0:1] + c1[:, 0:1], 1.0)
        t = (z0[...] + z1[...]) / cnt + zr_ref[...]
        m = jnp.mean(t)
        v = jnp.mean((t - m) ** 2)
        o_ref[...] = (t - m) / jnp.sqrt(v + 1e-5) * g_ref[...] + b_ref[...]

    return pl.pallas_call(
        body,
        out_shape=jax.ShapeDtypeStruct((N, D), jnp.float32),
    )(aggz[:N], aggz[N:], aggc[:N], aggc[N:], zr, g, b)


def _sc_sage(y, src, dst, with_count):
    mesh = plsc.VectorSubcoreMesh(core_axis_name="c", subcore_axis_name="s", num_cores=NC, num_subcores=NS)
    nout = 2 if with_count else 1
    scratch = [
        pltpu.VMEM((_CS,), jnp.int32),
        pltpu.VMEM((_CS,), jnp.int32),
        pltpu.VMEM((_CS, D), jnp.float32),
        pltpu.VMEM((_DC, D), jnp.float32),
        pltpu.VMEM_SHARED((N, D), jnp.float32),
    ]
    if with_count:
        scratch += [
            pltpu.VMEM((_CS, D), jnp.float32),
            pltpu.VMEM_SHARED((N, D), jnp.float32),
        ]

    @functools.partial(
        pl.kernel,
        out_type=[jax.ShapeDtypeStruct((NC * N, D), jnp.float32)] * nout,
        mesh=mesh,
        scratch_types=scratch,
        compiler_params=pltpu.CompilerParams(use_tc_tiling_on_sc=False, needs_layout_passes=False),
    )
    def kfn(y_hbm, src_hbm, dst_hbm, *rest):
        if with_count:
            outy, outc, si, di, yb, db, accy, cb, accc = rest
        else:
            outy, si, di, yb, db, accy = rest
            outc = cb = accc = None
        c = lax.axis_index("c")
        s = lax.axis_index("s")
        iot = lax.iota(jnp.int32, D)
        st = s * 624 + jnp.minimum(s, 2) * 8

        def zrow(i, _):
            db[i, pl.ds(0, D)] = jnp.zeros((D,), jnp.float32)
            return 0

        lax.fori_loop(0, _DC, zrow, 0)

        def zcp(g, _):
            r0 = st + g * _DC
            pltpu.sync_copy(db, accy.at[pl.ds(r0, _DC)])
            if with_count:
                pltpu.sync_copy(db, accc.at[pl.ds(r0, _DC)])
            return 0

        lax.fori_loop(0, 3, zcp, 0)

        @pl.when(s < 2)
        def _():
            r0 = st + 3 * _DC
            pltpu.sync_copy(db.at[pl.ds(0, 8)], accy.at[pl.ds(r0, 8)])
            if with_count:
                pltpu.sync_copy(db.at[pl.ds(0, 8)], accc.at[pl.ds(r0, 8)])

        if with_count:
            onec = jnp.where(iot == 0, 1.0, 0.0).astype(jnp.float32)

            def prep(e, _):
                cb[e, pl.ds(0, D)] = onec
                return 0

            lax.fori_loop(0, _CS, prep, 0)
        plsc.subcore_barrier()

        wid = c * NS + s

        def chunk(i, _):
            base = wid * (E // (NC * NS)) + i * _CS
            pltpu.sync_copy(src_hbm.at[pl.ds(base, _CS)], si)
            pltpu.sync_copy(dst_hbm.at[pl.ds(base, _CS)], di)
            pltpu.sync_copy(y_hbm.at[si], yb)
            pltpu.sync_copy(yb, accy.at[di], add=True)
            if with_count:
                pltpu.sync_copy(cb, accc.at[di], add=True)
            return 0

        lax.fori_loop(0, _NCHS, chunk, 0)
        plsc.subcore_barrier()

        def drain(g, _):
            r0 = st + g * _DC
            pltpu.sync_copy(accy.at[pl.ds(r0, _DC)], outy.at[pl.ds(c * N + r0, _DC)])
            if with_count:
                pltpu.sync_copy(accc.at[pl.ds(r0, _DC)], outc.at[pl.ds(c * N + r0, _DC)])
            return 0

        lax.fori_loop(0, 3, drain, 0)

        @pl.when(s < 2)
        def _():
            r0 = st + 3 * _DC
            pltpu.sync_copy(accy.at[pl.ds(r0, 8)], outy.at[pl.ds(c * N + r0, 8)])
            if with_count:
                pltpu.sync_copy(accc.at[pl.ds(r0, 8)], outc.at[pl.ds(c * N + r0, 8)])

    return kfn(y, src, dst)


def _sc_attn(q, k, v, ei):
    mesh = plsc.VectorSubcoreMesh(core_axis_name="c", subcore_axis_name="s", num_cores=NC, num_subcores=NS)

    nset = 2
    per_set = [
        pltpu.VMEM((2, _CA), jnp.int32),
        pltpu.VMEM((_CA,), jnp.int32),
        pltpu.VMEM((_CA,), jnp.int32),
        pltpu.VMEM((_CA,), jnp.int32),
        pltpu.VMEM((_CA, 128), jnp.float32),
        pltpu.VMEM((_CA, 128), jnp.float32),
        pltpu.VMEM((_CA, 128), jnp.float32),
        pltpu.VMEM((_CA, 144), jnp.float32),
        pltpu.SemaphoreType.DMA,
        pltpu.SemaphoreType.DMA,
    ]

    @functools.partial(
        pl.kernel,
        out_type=jax.ShapeDtypeStruct((NC * N, 144), jnp.float32),
        mesh=mesh,
        scratch_types=per_set * nset + [
            pltpu.VMEM_SHARED((N + 16, 144), jnp.float32),
        ],
        compiler_params=pltpu.CompilerParams(use_tc_tiling_on_sc=False, needs_layout_passes=False),
    )
    def kfn(q_hbm, k_hbm, v_hbm, ei_hbm, out_hbm, *scr):
        sets = [scr[i * 10:(i + 1) * 10] for i in range(nset)]
        acc = scr[nset * 10]
        c = lax.axis_index("c")
        s = lax.axis_index("s")
        cN = c * N
        iot = lax.iota(jnp.int32, 16)
        st = s * 624 + jnp.minimum(s, 2) * 8
        stg0 = sets[0][7]

        def zrow(i, _):
            for j in range(9):
                stg0[i, pl.ds(j * 16, 16)] = jnp.zeros((16,), jnp.float32)
            return 0

        lax.fori_loop(0, _CA, zrow, 0)

        def zcp(g, _):
            pltpu.sync_copy(stg0, acc.at[pl.ds(st + g * _CA, _CA)])
            return 0

        lax.fori_loop(0, 19, zcp, 0)
        pltpu.sync_copy(stg0.at[pl.ds(0, 16)], acc.at[pl.ds(st + 608, 16)])

        @pl.when(s < 2)
        def _():
            pltpu.sync_copy(stg0.at[pl.ds(0, 8)], acc.at[pl.ds(st + 624, 8)])

        @pl.when(s == 0)
        def _():
            pltpu.sync_copy(stg0.at[pl.ds(0, 16)], acc.at[pl.ds(N, 16)])

        plsc.subcore_barrier()

        cbase = s * _NCHA

        def load_and_fire(i, st_):
            eb, gsk, gq = st_[0], st_[1], st_[2]
            qb, kb, vb, gsem = st_[4], st_[5], st_[6], st_[8]
            pltpu.sync_copy(ei_hbm.at[cbase + i], eb)
            for j in range(_CA // 16):
                sl = pl.ds(j * 16, 16)
                gsk[sl] = eb[0, sl] + cN
                gq[sl] = jnp.minimum(eb[1, sl], N - 1) + cN
            pltpu.async_copy(q_hbm.at[gq], qb, gsem)
            pltpu.async_copy(k_hbm.at[gsk], kb, gsem)
            pltpu.async_copy(v_hbm.at[gsk], vb, gsem)

        def run_chunk(i, g, st_, st_next):
            eb, gsk, gq, dsc, qb, kb, vb, stg, gsem, ssem = st_
            pltpu.make_async_copy(q_hbm.at[gq], qb, gsem).wait()
            pltpu.make_async_copy(k_hbm.at[gsk], kb, gsem).wait()
            pltpu.make_async_copy(v_hbm.at[gsk], vb, gsem).wait()

            @pl.when(i + 1 < _NCHA)
            def _():
                load_and_fire(i + 1, st_next)

            @pl.when(g > 0)
            def _():
                pltpu.make_async_copy(stg, acc.at[dsc], ssem).wait()

            for j in range(_CA // 16):
                sl = pl.ds(j * 16, 16)
                dsc[sl] = eb[1, sl]

            @plsc.parallel_loop(0, _CA, step=1, unroll=8)
            def edge(e):
                lgv = jnp.zeros((16,), jnp.float32)
                for j in range(8):
                    sl = pl.ds(j * 16, 16)
                    lg = jnp.sum(qb[e, sl] * kb[e, sl])
                    lgv = jnp.where(iot == j, lg, lgv)
                exv = jnp.exp(lgv)
                for j in range(8):
                    sl = pl.ds(j * 16, 16)
                    stg[e, sl] = vb[e, sl] * exv[j]
                stg[e, pl.ds(128, 16)] = exv

            pltpu.async_copy(stg, acc.at[dsc], ssem, add=True)

        load_and_fire(0, sets[0])

        def pair(g, _):
            run_chunk(2 * g, g, sets[0], sets[1])
            run_chunk(2 * g + 1, g, sets[1], sets[0])
            return 0

        lax.fori_loop(0, _NCHA // 2, pair, 0)
        for b in range(nset):
            dsc, stg, ssem = sets[b][3], sets[b][7], sets[b][9]
            pltpu.make_async_copy(stg, acc.at[dsc], ssem).wait()
        plsc.subcore_barrier()

        def drain(g, _):
            r0 = st + g * _DC
            pltpu.sync_copy(acc.at[pl.ds(r0, _DC)], out_hbm.at[pl.ds(cN + r0, _DC)])
            return 0

        lax.fori_loop(0, 3, drain, 0)

        @pl.when(s < 2)
        def _():
            r0 = st + 3 * _DC
            pltpu.sync_copy(acc.at[pl.ds(r0, 8)], out_hbm.at[pl.ds(cN + r0, 8)])

    return kfn(q, k, v, ei)


def kernel(x, llc_x, edge_index, params):
    p = params
    src = edge_index[0].astype(jnp.int32)
    dst = edge_index[1].astype(jnp.int32)
    pad = _EP - E
    src_p = jnp.concatenate([src, jnp.zeros((pad,), jnp.int32)])
    dst_p = jnp.concatenate([dst, jnp.full((pad,), N, jnp.int32)])
    ei = jnp.stack([src_p.reshape(-1, _CA), dst_p.reshape(-1, _CA)], axis=1)

    def b2(a):
        return a.reshape(1, -1)

    y, xr = _tc_embed(x, p['emb_Wl'], p['emb_Wr'])
    aggy, aggc = _sc_sage(y, src, dst, with_count=True)
    a1 = p['a1']
    q1, k1, v1, s1 = _tc_qkv1(
        aggy, aggc, xr, b2(p['emb_bl']),
        a1['Wq'], b2(a1['bq']), a1['Wk'], b2(a1['bk']),
        a1['Wv'], b2(a1['bv']), a1['Ws'], b2(a1['bs']))
    att1 = _sc_attn(q1.reshape(NC * N, 128), k1.reshape(NC * N, 128),
                    v1.reshape(NC * N, 128), ei)
    t_pre = _tc_cat(att1, s1, p['cat1_W'], b2(p['cat1_b']))
    t = _tc_ln(t_pre, b2(p['ln1_g']), b2(p['ln1_b']))

    a2 = p['a2']
    q2, k2, v2, s2 = _tc_qkv2(
        t, llc_x,
        a2['Wq'], b2(a2['bq']), a2['Wk'], b2(a2['bk']),
        a2['Wv'], b2(a2['bv']), a2['Ws'], b2(a2['bs']))
    att2 = _sc_attn(q2.reshape(NC * N, 128), k2.reshape(NC * N, 128),
                    v2.reshape(NC * N, 128), ei)
    t2_pre = _tc_cat(att2, s2, p['cat2_W'], b2(p['cat2_b']))
    z, zr = _tc_lnff(t2_pre, b2(p['ln2_g']), b2(p['ln2_b']),
                     p['ff_Wl'], p['ff_Wr'], b2(p['ff_bl']))

    (aggz,) = _sc_sage(z, src, dst, with_count=False)
    return _tc_final(aggz, aggc, zr, b2(p['ln3_g']), b2(p['ln3_b']))


# speedup vs baseline: 36.1200x; 1.2469x over previous
import functools

import jax
import jax.numpy as jnp
from jax import lax
from jax.experimental import pallas as pl
from jax.experimental.pallas import tpu as pltpu
from jax.experimental.pallas import tpu_sc as plsc

N = 10000
E = 160000
D_IN = 128
D = 16
H = 16
HD = 256
NC = 2
NS = 16
LPT = N // NS
RB = N // 5

_CA = 32
_EP = 160768
_NCHA = (_EP // NS) // _CA
_CS = 1000
_DC = 208
_NCHS = (E // (NC * NS)) // _CS


def _tc_embed(x, Wl, Wr):
    def body(x_ref, wl_ref, wr_ref, y_ref, xr_ref):
        xb = x_ref[...]
        y_ref[...] = jnp.dot(xb, wl_ref[...], preferred_element_type=jnp.float32)
        xr_ref[...] = jnp.dot(xb, wr_ref[...], preferred_element_type=jnp.float32)

    return pl.pallas_call(
        body,
        grid=(5,),
        in_specs=[
            pl.BlockSpec((RB, D_IN), lambda i: (i, 0)),
            pl.BlockSpec((D_IN, D), lambda i: (0, 0)),
            pl.BlockSpec((D_IN, D), lambda i: (0, 0)),
        ],
        out_specs=[
            pl.BlockSpec((RB, D), lambda i: (i, 0)),
            pl.BlockSpec((RB, D), lambda i: (i, 0)),
        ],
        out_shape=[jax.ShapeDtypeStruct((N, D), jnp.float32)] * 2,
    )(x, Wl, Wr)


def _tc_qkv1(aggy, aggc, xr, bl, Wq, bq, Wk, bk, Wv, bv, Ws, bs):
    def body(a0, a1, c0, c1, xr_ref, bl_ref, wq, bq_, wk, bk_, wv, bv_,
             ws, bs_, oq, ok, ov, os_):
        cnt = jnp.maximum(c0[:, 0:1] + c1[:, 0:1], 1.0)
        h = (a0[...] + a1[...]) / cnt + bl_ref[...] + xr_ref[...]
        q = (jnp.dot(h, wq[...], preferred_element_type=jnp.float32)
             + bq_[...]) * 0.25
        k = jnp.dot(h, wk[...], preferred_element_type=jnp.float32) + bk_[...]
        v = jnp.dot(h, wv[...], preferred_element_type=jnp.float32) + bv_[...]
        oq[0] = q[:, :128]
        oq[1] = q[:, 128:]
        ok[0] = k[:, :128]
        ok[1] = k[:, 128:]
        ov[0] = v[:, :128]
        ov[1] = v[:, 128:]
        os_[...] = jnp.dot(h, ws[...], preferred_element_type=jnp.float32) + bs_[...]

    half = pl.BlockSpec((RB, D), lambda i: (i, 0))
    wspec = pl.BlockSpec((D, HD), lambda i: (0, 0))
    bspec = pl.BlockSpec((1, HD), lambda i: (0, 0))
    big = pl.BlockSpec((2, RB, 128), lambda i: (0, i, 0))
    return pl.pallas_call(
        body,
        grid=(5,),
        in_specs=[half, half, half, half, half,
                  pl.BlockSpec((1, D), lambda i: (0, 0)),
                  wspec, bspec, wspec, bspec, wspec, bspec, wspec, bspec],
        out_specs=[big, big, big, pl.BlockSpec((RB, HD), lambda i: (i, 0))],
        out_shape=[jax.ShapeDtypeStruct((2, N, 128), jnp.float32)] * 3
        + [jax.ShapeDtypeStruct((N, HD), jnp.float32)],
    )(aggy[:N], aggy[N:], aggc[:N], aggc[N:], xr, bl,
      Wq, bq, Wk, bk, Wv, bv, Ws, bs)


def _tc_qkv2(t, llc, Wq, bq, Wk, bk, Wv, bv, Ws, bs):
    def body(t_ref, l_ref, wq, bq_, wk, bk_, wv, bv_, ws, bs_,
             oq, ok, ov, os_):
        tb = t_ref[...]
        lb = l_ref[...]
        q = (jnp.dot(lb, wq[...], preferred_element_type=jnp.float32)
             + bq_[...]) * 0.25
        k = jnp.dot(tb, wk[...], preferred_element_type=jnp.float32) + bk_[...]
        v = jnp.dot(tb, wv[...], preferred_element_type=jnp.float32) + bv_[...]
        oq[0] = q[:, :128]
        oq[1] = q[:, 128:]
        ok[0] = k[:, :128]
        ok[1] = k[:, 128:]
        ov[0] = v[:, :128]
        ov[1] = v[:, 128:]
        os_[...] = jnp.dot(lb, ws[...], preferred_element_type=jnp.float32) + bs_[...]

    half = pl.BlockSpec((RB, D), lambda i: (i, 0))
    wspec = pl.BlockSpec((D, HD), lambda i: (0, 0))
    bspec = pl.BlockSpec((1, HD), lambda i: (0, 0))
    big = pl.BlockSpec((2, RB, 128), lambda i: (0, i, 0))
    return pl.pallas_call(
        body,
        grid=(5,),
        in_specs=[half, half, wspec, bspec, wspec, bspec, wspec, bspec,
                  wspec, bspec],
        out_specs=[big, big, big, pl.BlockSpec((RB, HD), lambda i: (i, 0))],
        out_shape=[jax.ShapeDtypeStruct((2, N, 128), jnp.float32)] * 3
        + [jax.ShapeDtypeStruct((N, HD), jnp.float32)],
    )(t, llc, Wq, bq, Wk, bk, Wv, bv, Ws, bs)


def _tc_cat(att, s, W, b):
    def body(a0, a1, s_ref, w_ref, b_ref, o_ref):
        cols = []
        for a in (a0[...], a1[...]):
            for j in range(8):
                den = a[:, 128 + j:129 + j] + 1e-16
                cols.append(a[:, j * 16:(j + 1) * 16] / den)
        tf = jnp.concatenate(cols, axis=1) + s_ref[...]
        o_ref[...] = (
            jnp.dot(tf, w_ref[...], preferred_element_type=jnp.float32)
            + b_ref[...]
        )

    return pl.pallas_call(
        body,
        grid=(5,),
        in_specs=[
            pl.BlockSpec((RB, 144), lambda i: (i, 0)),
            pl.BlockSpec((RB, 144), lambda i: (i + 5, 0)),
            pl.BlockSpec((RB, HD), lambda i: (i, 0)),
            pl.BlockSpec((HD, D), lambda i: (0, 0)),
            pl.BlockSpec((1, D), lambda i: (0, 0)),
        ],
        out_specs=pl.BlockSpec((RB, D), lambda i: (i, 0)),
        out_shape=jax.ShapeDtypeStruct((N, D), jnp.float32),
    )(att, att, s, W, b)


def _tc_ln(t_pre, g, b):
    def body(t_ref, g_ref, b_ref, o_ref):
        t = t_ref[...]
        m = jnp.mean(t)
        v = jnp.mean((t - m) ** 2)
        o_ref[...] = (t - m) / jnp.sqrt(v + 1e-5) * g_ref[...] + b_ref[...]

    return pl.pallas_call(
        body,
        out_shape=jax.ShapeDtypeStruct((N, D), jnp.float32),
    )(t_pre, g, b)


def _tc_lnff(t2_pre, g, b, Wl, Wr, bl):
    def body(t_ref, g_ref, b_ref, wl, wr, bl_, oz, ozr):
        t = t_ref[...]
        m = jnp.mean(t)
        v = jnp.mean((t - m) ** 2)
        t2 = (t - m) / jnp.sqrt(v + 1e-5) * g_ref[...] + b_ref[...]
        oz[...] = jnp.dot(t2, wl[...], preferred_element_type=jnp.float32)
        ozr[...] = (
            jnp.dot(t2, wr[...], preferred_element_type=jnp.float32) + bl_[...]
        )

    return pl.pallas_call(
        body,
        out_shape=[jax.ShapeDtypeStruct((N, D), jnp.float32)] * 2,
    )(t2_pre, g, b, Wl, Wr, bl)


def _tc_final(aggz, aggc, zr, g, b):
    def body(z0, z1, c0, c1, zr_ref, g_ref, b_ref, o_ref):
        cnt = jnp.maximum(c0[:, 0:1] + c1[:, 0:1], 1.0)
        t = (z0[...] + z1[...]) / cnt + zr_ref[...]
        m = jnp.mean(t)
        v = jnp.mean((t - m) ** 2)
        o_ref[...] = (t - m) / jnp.sqrt(v + 1e-5) * g_ref[...] + b_ref[...]

    return pl.pallas_call(
        body,
        out_shape=jax.ShapeDtypeStruct((N, D), jnp.float32),
    )(aggz[:N], aggz[N:], aggc[:N], aggc[N:], zr, g, b)


def _sc_sage(y, src, dst, with_count):
    mesh = plsc.VectorSubcoreMesh(core_axis_name="c", subcore_axis_name="s", num_cores=NC, num_subcores=NS)
    nout = 2 if with_count else 1
    scratch = [
        pltpu.VMEM((_CS,), jnp.int32),
        pltpu.VMEM((_CS,), jnp.int32),
        pltpu.VMEM((_CS, D), jnp.float32),
        pltpu.VMEM((_DC, D), jnp.float32),
        pltpu.VMEM_SHARED((N, D), jnp.float32),
    ]
    if with_count:
        scratch += [
            pltpu.VMEM((_CS, D), jnp.float32),
            pltpu.VMEM_SHARED((N, D), jnp.float32),
        ]

    @functools.partial(
        pl.kernel,
        out_type=[jax.ShapeDtypeStruct((NC * N, D), jnp.float32)] * nout,
        mesh=mesh,
        scratch_types=scratch,
        compiler_params=pltpu.CompilerParams(use_tc_tiling_on_sc=False, needs_layout_passes=False),
    )
    def kfn(y_hbm, src_hbm, dst_hbm, *rest):
        if with_count:
            outy, outc, si, di, yb, db, accy, cb, accc = rest
        else:
            outy, si, di, yb, db, accy = rest
            outc = cb = accc = None
        c = lax.axis_index("c")
        s = lax.axis_index("s")
        iot = lax.iota(jnp.int32, D)
        st = s * 624 + jnp.minimum(s, 2) * 8

        def zrow(i, _):
            db[i, pl.ds(0, D)] = jnp.zeros((D,), jnp.float32)
            return 0

        lax.fori_loop(0, _DC, zrow, 0)

        def zcp(g, _):
            r0 = st + g * _DC
            pltpu.sync_copy(db, accy.at[pl.ds(r0, _DC)])
            if with_count:
                pltpu.sync_copy(db, accc.at[pl.ds(r0, _DC)])
            return 0

        lax.fori_loop(0, 3, zcp, 0)

        @pl.when(s < 2)
        def _():
            r0 = st + 3 * _DC
            pltpu.sync_copy(db.at[pl.ds(0, 8)], accy.at[pl.ds(r0, 8)])
            if with_count:
                pltpu.sync_copy(db.at[pl.ds(0, 8)], accc.at[pl.ds(r0, 8)])

        if with_count:
            onec = jnp.where(iot == 0, 1.0, 0.0).astype(jnp.float32)

            def prep(e, _):
                cb[e, pl.ds(0, D)] = onec
                return 0

            lax.fori_loop(0, _CS, prep, 0)
        plsc.subcore_barrier()

        wid = c * NS + s

        def chunk(i, _):
            base = wid * (E // (NC * NS)) + i * _CS
            pltpu.sync_copy(src_hbm.at[pl.ds(base, _CS)], si)
            pltpu.sync_copy(dst_hbm.at[pl.ds(base, _CS)], di)
            pltpu.sync_copy(y_hbm.at[si], yb)
            pltpu.sync_copy(yb, accy.at[di], add=True)
            if with_count:
                pltpu.sync_copy(cb, accc.at[di], add=True)
            return 0

        lax.fori_loop(0, _NCHS, chunk, 0)
        plsc.subcore_barrier()

        def drain(g, _):
            r0 = st + g * _DC
            pltpu.sync_copy(accy.at[pl.ds(r0, _DC)], outy.at[pl.ds(c * N + r0, _DC)])
            if with_count:
                pltpu.sync_copy(accc.at[pl.ds(r0, _DC)], outc.at[pl.ds(c * N + r0, _DC)])
            return 0

        lax.fori_loop(0, 3, drain, 0)

        @pl.when(s < 2)
        def _():
            r0 = st + 3 * _DC
            pltpu.sync_copy(accy.at[pl.ds(r0, 8)], outy.at[pl.ds(c * N + r0, 8)])
            if with_count:
                pltpu.sync_copy(accc.at[pl.ds(r0, 8)], outc.at[pl.ds(c * N + r0, 8)])

    return kfn(y, src, dst)


def _sc_attn(q, k, v, ei):
    mesh = plsc.VectorSubcoreMesh(core_axis_name="c", subcore_axis_name="s", num_cores=NC, num_subcores=NS)

    nset = 2
    per_set = [
        pltpu.VMEM((2, _CA), jnp.int32),
        pltpu.VMEM((_CA,), jnp.int32),
        pltpu.VMEM((_CA,), jnp.int32),
        pltpu.VMEM((_CA,), jnp.int32),
        pltpu.VMEM((_CA, 128), jnp.float32),
        pltpu.VMEM((_CA, 128), jnp.float32),
        pltpu.VMEM((_CA, 128), jnp.float32),
        pltpu.VMEM((_CA, 144), jnp.float32),
        pltpu.SemaphoreType.DMA,
        pltpu.SemaphoreType.DMA,
    ]

    @functools.partial(
        pl.kernel,
        out_type=jax.ShapeDtypeStruct((NC * N, 144), jnp.float32),
        mesh=mesh,
        scratch_types=per_set * nset + [
            pltpu.VMEM_SHARED((N + 16, 144), jnp.float32),
        ],
        compiler_params=pltpu.CompilerParams(use_tc_tiling_on_sc=False, needs_layout_passes=False),
    )
    def kfn(q_hbm, k_hbm, v_hbm, ei_hbm, out_hbm, *scr):
        sets = [scr[i * 10:(i + 1) * 10] for i in range(nset)]
        acc = scr[nset * 10]
        c = lax.axis_index("c")
        s = lax.axis_index("s")
        cN = c * N
        iot = lax.iota(jnp.int32, 16)
        st = s * 624 + jnp.minimum(s, 2) * 8
        stg0 = sets[0][7]

        def zrow(i, _):
            for j in range(9):
                stg0[i, pl.ds(j * 16, 16)] = jnp.zeros((16,), jnp.float32)
            return 0

        lax.fori_loop(0, _CA, zrow, 0)

        def zcp(g, _):
            pltpu.sync_copy(stg0, acc.at[pl.ds(st + g * _CA, _CA)])
            return 0

        lax.fori_loop(0, 19, zcp, 0)
        pltpu.sync_copy(stg0.at[pl.ds(0, 16)], acc.at[pl.ds(st + 608, 16)])

        @pl.when(s < 2)
        def _():
            pltpu.sync_copy(stg0.at[pl.ds(0, 8)], acc.at[pl.ds(st + 624, 8)])

        @pl.when(s == 0)
        def _():
            pltpu.sync_copy(stg0.at[pl.ds(0, 16)], acc.at[pl.ds(N, 16)])

        plsc.subcore_barrier()

        cbase = s * _NCHA

        def load_and_fire(i, st_):
            eb, gsk, gq = st_[0], st_[1], st_[2]
            qb, kb, vb, gsem = st_[4], st_[5], st_[6], st_[8]
            pltpu.sync_copy(ei_hbm.at[cbase + i], eb)
            for j in range(_CA // 16):
                sl = pl.ds(j * 16, 16)
                gsk[sl] = eb[0, sl] + cN
                gq[sl] = jnp.minimum(eb[1, sl], N - 1) + cN
            pltpu.async_copy(q_hbm.at[gq], qb, gsem)
            pltpu.async_copy(k_hbm.at[gsk], kb, gsem)
            pltpu.async_copy(v_hbm.at[gsk], vb, gsem)

        def run_chunk(i, g, st_, st_next):
            eb, gsk, gq, dsc, qb, kb, vb, stg, gsem, ssem = st_
            pltpu.make_async_copy(q_hbm.at[gq], qb, gsem).wait()
            pltpu.make_async_copy(k_hbm.at[gsk], kb, gsem).wait()
            pltpu.make_async_copy(v_hbm.at[gsk], vb, gsem).wait()

            @pl.when(i + 1 < _NCHA)
            def _():
                load_and_fire(i + 1, st_next)

            @pl.when(g > 0)
            def _():
                pltpu.make_async_copy(stg, acc.at[dsc], ssem).wait()

            for j in range(_CA // 16):
                sl = pl.ds(j * 16, 16)
                dsc[sl] = eb[1, sl]

            @plsc.parallel_loop(0, _CA, step=1, unroll=8)
            def edge(e):
                lgv = jnp.zeros((16,), jnp.float32)
                for j in range(8):
                    sl = pl.ds(j * 16, 16)
                    lg = jnp.sum(qb[e, sl] * kb[e, sl])
                    lgv = jnp.where(iot == j, lg, lgv)
                exv = jnp.exp(lgv)
                for j in range(8):
                    sl = pl.ds(j * 16, 16)
                    stg[e, sl] = vb[e, sl] * exv[j]
                stg[e, pl.ds(128, 16)] = exv

            pltpu.async_copy(stg, acc.at[dsc], ssem, add=True)

        load_and_fire(0, sets[0])

        def pair(g, _):
            run_chunk(2 * g, g, sets[0], sets[1])
            run_chunk(2 * g + 1, g, sets[1], sets[0])
            return 0

        lax.fori_loop(0, _NCHA // 2, pair, 0)
        for b in range(nset):
            dsc, stg, ssem = sets[b][3], sets[b][7], sets[b][9]
            pltpu.make_async_copy(stg, acc.at[dsc], ssem).wait()
        plsc.subcore_barrier()

        def drain(g, _):
            r0 = st + g * _DC
            pltpu.sync_copy(acc.at[pl.ds(r0, _DC)], out_hbm.at[pl.ds(cN + r0, _DC)])
            return 0

        lax.fori_loop(0, 3, drain, 0)

        @pl.when(s < 2)
        def _():
            r0 = st + 3 * _DC
            pltpu.sync_copy(acc.at[pl.ds(r0, 8)], out_hbm.at[pl.ds(cN + r0, 8)])

    return kfn(q, k, v, ei)


def kernel(x, llc_x, edge_index, params):
    p = params
    src = edge_index[0].astype(jnp.int32)
    dst = edge_index[1].astype(jnp.int32)
    pad = _EP - E
    src_p = jnp.concatenate([src, jnp.zeros((pad,), jnp.int32)])
    dst_p = jnp.concatenate([dst, jnp.full((pad,), N, jnp.int32)])
    ei = jnp.stack([src_p.reshape(-1, _CA), dst_p.reshape(-1, _CA)], axis=1)

    def b2(a):
        return a.reshape(1, -1)

    y, xr = _tc_embed(x, p['emb_Wl'], p['emb_Wr'])
    aggy, aggc = _sc_sage(y, src, dst, with_count=True)
    a1 = p['a1']
    q1, k1, v1, s1 = _tc_qkv1(
        aggy, aggc, xr, b2(p['emb_bl']),
        a1['Wq'], b2(a1['bq']), a1['Wk'], b2(a1['bk']),
        a1['Wv'], b2(a1['bv']), a1['Ws'], b2(a1['bs']))
    att1 = _sc_attn(q1.reshape(NC * N, 128), k1.reshape(NC * N, 128),
                    v1.reshape(NC * N, 128), ei)
    t_pre = _tc_cat(att1, s1, p['cat1_W'], b2(p['cat1_b']))
    t = _tc_ln(t_pre, b2(p['ln1_g']), b2(p['ln1_b']))

    a2 = p['a2']
    q2, k2, v2, s2 = _tc_qkv2(
        t, llc_x,
        a2['Wq'], b2(a2['bq']), a2['Wk'], b2(a2['bk']),
        a2['Wv'], b2(a2['bv']), a2['Ws'], b2(a2['bs']))
    att2 = _sc_attn(q2.reshape(NC * N, 128), k2.reshape(NC * N, 128),
                    v2.reshape(NC * N, 128), ei)
    t2_pre = _tc_cat(att2, s2, p['cat2_W'], b2(p['cat2_b']))
    z, zr = _tc_lnff(t2_pre, b2(p['ln2_g']), b2(p['ln2_b']),
                     p['ff_Wl'], p['ff_Wr'], b2(p['ff_bl']))

    (aggz,) = _sc_sage(z, src, dst, with_count=False)
    return _tc_final(aggz, aggc, zr, b2(p['ln3_g']), b2(p['ln3_b']))
